# Initial kernel scaffold; baseline (speedup 1.0000x reference)
import jax
import jax.numpy as jnp
from jax.experimental import pallas as pl


def kernel(x, edge_index, W_enc, b_enc, W1, b1, g1, bt1, W2, b2, g2, bt2, W3, b3, g3, bt3, Wf1, bf1, gf1, btf1, Wf2, bf2):
    raise NotImplementedError("write your pallas kernel here")


# speedup vs baseline: 10.0016x; 10.0016x over previous
import functools
import math

import jax
import jax.numpy as jnp
from jax import lax
from jax.experimental import pallas as pl
from jax.experimental.pallas import tpu as pltpu
from jax.experimental.pallas import tpu_sc as plsc

NC = 2
NS = 16
CH = 128
BN_SCALE = 1.0 / math.sqrt(1.0 + 1e-5)


def _deg_body(np_, rps, cpw, dst_ref, out_ref, acc, idx_d, ones, zbuf, sem):
    c = lax.axis_index("c")
    s = lax.axis_index("s")
    w = c * NS + s

    def fill_z(i, _):
        zbuf[i, :] = jnp.zeros((16,), jnp.float32)
        return 0

    lax.fori_loop(0, rps, fill_z, 0)

    def fill_o(i, _):
        ones[i, :] = jnp.ones((16,), jnp.float32)
        return 0

    lax.fori_loop(0, CH, fill_o, 0)

    pltpu.sync_copy(zbuf, acc.at[pl.ds(s * rps, rps)])
    plsc.subcore_barrier()

    def step(i, _):
        ch = w * cpw + i
        pltpu.sync_copy(dst_ref.at[pl.ds(ch * CH, CH)], idx_d)
        pltpu.sync_copy(ones, acc.at[idx_d], add=True)
        return 0

    lax.fori_loop(0, cpw, step, 0)
    plsc.subcore_barrier()
    pltpu.sync_copy(acc.at[pl.ds(s * rps, rps)],
                    out_ref.at[pl.ds(c * np_ + s * rps, rps)])


def _scatter_body(rps, cps, lo_ref, hi_ref, src_ref, dst_ref,
                  out_lo_ref, out_hi_ref, acc, idx_s, idx_d, rows, sem):
    c = lax.axis_index("c")
    s = lax.axis_index("s")

    def run(hs_ref, out_ref):
        pltpu.sync_copy(hs_ref.at[pl.ds(s * rps, rps)],
                        acc.at[pl.ds(s * rps, rps)])
        plsc.subcore_barrier()

        def step(i, _):
            ch = s * cps + i
            pltpu.sync_copy(src_ref.at[pl.ds(ch * CH, CH)], idx_s)
            pltpu.sync_copy(dst_ref.at[pl.ds(ch * CH, CH)], idx_d)
            pltpu.async_copy(hs_ref.at[idx_s], rows, sem).wait()
            pltpu.sync_copy(rows, acc.at[idx_d], add=True)
            return 0

        lax.fori_loop(0, cps, step, 0)
        plsc.subcore_barrier()
        pltpu.sync_copy(acc.at[pl.ds(s * rps, rps)],
                        out_ref.at[pl.ds(s * rps, rps)])

    pl.when(c == 0)(lambda: run(lo_ref, out_lo_ref))
    pl.when(c == 1)(lambda: run(hi_ref, out_hi_ref))


@functools.partial(jax.jit, static_argnames=("np_", "rps", "cpw"))
def _sc_degree(dst, *, np_, rps, cpw):
    mesh = plsc.VectorSubcoreMesh(core_axis_name="c", subcore_axis_name="s")
    body = functools.partial(_deg_body, np_, rps, cpw)
    return pl.kernel(
        body,
        out_type=jax.ShapeDtypeStruct((NC * np_, 16), jnp.float32),
        mesh=mesh,
        scratch_types=[
            pltpu.VMEM_SHARED((np_, 16), jnp.float32),
            pltpu.VMEM((CH,), jnp.int32),
            pltpu.VMEM((CH, 16), jnp.float32),
            pltpu.VMEM((rps, 16), jnp.float32),
            pltpu.SemaphoreType.DMA,
        ],
        compiler_params=pltpu.CompilerParams(use_tc_tiling_on_sc=False),
        name="sc_gcn_degree",
    )(dst)


@functools.partial(jax.jit, static_argnames=("np_", "rps", "cps"))
def _sc_aggregate(hs_lo, hs_hi, src, dst, *, np_, rps, cps):
    mesh = plsc.VectorSubcoreMesh(core_axis_name="c", subcore_axis_name="s")
    body = functools.partial(_scatter_body, rps, cps)
    return pl.kernel(
        body,
        out_type=(jax.ShapeDtypeStruct((np_, 32), jnp.float32),
                  jax.ShapeDtypeStruct((np_, 32), jnp.float32)),
        mesh=mesh,
        scratch_types=[
            pltpu.VMEM_SHARED((np_, 32), jnp.float32),
            pltpu.VMEM((CH,), jnp.int32),
            pltpu.VMEM((CH,), jnp.int32),
            pltpu.VMEM((CH, 32), jnp.float32),
            pltpu.SemaphoreType.DMA,
        ],
        compiler_params=pltpu.CompilerParams(use_tc_tiling_on_sc=False),
        name="sc_gcn_aggregate",
    )(hs_lo, hs_hi, src, dst)


def _pre_body(x_ref, p0_ref, p1_ref, We_ref, be_ref, W1_ref,
              h0_ref, lo_ref, hi_ref, dinv_ref):
    deg = 1.0 + p0_ref[:, :1] + p1_ref[:, :1]
    dinv = lax.rsqrt(deg)
    h0 = jax.nn.relu(jnp.dot(x_ref[:], We_ref[:],
                             preferred_element_type=jnp.float32) + be_ref[:])
    hs = dinv * jnp.dot(h0, W1_ref[:], preferred_element_type=jnp.float32)
    h0_ref[:] = h0
    lo_ref[:] = hs[:, :32]
    hi_ref[:] = hs[:, 32:]
    dinv_ref[:] = dinv


def _mid_body(lo_ref, hi_ref, dinv_ref, hprev_ref, b_ref, g_ref, bt_ref,
              Wn_ref, h_ref, nlo_ref, nhi_ref):
    accf = jnp.concatenate([lo_ref[:], hi_ref[:]], axis=1)
    dinv = dinv_ref[:]
    gcn = dinv * accf + b_ref[:]
    t = jax.nn.relu(g_ref[:] * (gcn * BN_SCALE) + bt_ref[:]) + hprev_ref[:]
    hs = dinv * jnp.dot(t, Wn_ref[:], preferred_element_type=jnp.float32)
    h_ref[:] = t
    nlo_ref[:] = hs[:, :32]
    nhi_ref[:] = hs[:, 32:]


def _final_body(lo_ref, hi_ref, dinv_ref, hprev_ref, b_ref, g_ref, bt_ref,
                Wf1_ref, bf1_ref, gf1_ref, btf1_ref, Wf2_ref, bf2_ref,
                out_ref):
    accf = jnp.concatenate([lo_ref[:], hi_ref[:]], axis=1)
    gcn = dinv_ref[:] * accf + b_ref[:]
    t = jax.nn.relu(g_ref[:] * (gcn * BN_SCALE) + bt_ref[:]) + hprev_ref[:]
    z = jnp.dot(t, Wf1_ref[:], preferred_element_type=jnp.float32) + bf1_ref[:]
    z = jax.nn.relu(gf1_ref[:] * (z * BN_SCALE) + btf1_ref[:])
    out_ref[:] = jnp.tanh(
        jnp.dot(z, Wf2_ref[:], preferred_element_type=jnp.float32) + bf2_ref[:])


def _row_spec(r, cols):
    return pl.BlockSpec((r, cols), lambda i: (i, 0))


def _full_spec(shape):
    return pl.BlockSpec(shape, lambda i: tuple(0 for _ in shape))


def _tc_pre(x, p0, p1, We, be, W1, *, n, np_, r):
    grid = (n // r,)
    return pl.pallas_call(
        _pre_body,
        grid=grid,
        in_specs=[_row_spec(r, 2), _row_spec(r, 16), _row_spec(r, 16),
                  _full_spec((2, 64)), _full_spec((1, 64)),
                  _full_spec((64, 64))],
        out_specs=[_row_spec(r, 64), _row_spec(r, 32), _row_spec(r, 32),
                   _row_spec(r, 1)],
        out_shape=[jax.ShapeDtypeStruct((n, 64), jnp.float32),
                   jax.ShapeDtypeStruct((np_, 32), jnp.float32),
                   jax.ShapeDtypeStruct((np_, 32), jnp.float32),
                   jax.ShapeDtypeStruct((n, 1), jnp.float32)],
        name="tc_gnn_pre",
    )(x, p0, p1, We, be, W1)


def _tc_mid(acc_lo, acc_hi, dinv, hprev, b, g, bt, Wn, *, n, np_, r):
    grid = (n // r,)
    return pl.pallas_call(
        _mid_body,
        grid=grid,
        in_specs=[_row_spec(r, 32), _row_spec(r, 32), _row_spec(r, 1),
                  _row_spec(r, 64), _full_spec((1, 64)), _full_spec((1, 64)),
                  _full_spec((1, 64)), _full_spec((64, 64))],
        out_specs=[_row_spec(r, 64), _row_spec(r, 32), _row_spec(r, 32)],
        out_shape=[jax.ShapeDtypeStruct((n, 64), jnp.float32),
                   jax.ShapeDtypeStruct((np_, 32), jnp.float32),
                   jax.ShapeDtypeStruct((np_, 32), jnp.float32)],
        name="tc_gnn_mid",
    )(acc_lo, acc_hi, dinv, hprev, b, g, bt, Wn)


def _tc_final(acc_lo, acc_hi, dinv, hprev, b, g, bt, Wf1, bf1, gf1, btf1,
              Wf2, bf2, *, n, r):
    grid = (n // r,)
    return pl.pallas_call(
        _final_body,
        grid=grid,
        in_specs=[_row_spec(r, 32), _row_spec(r, 32), _row_spec(r, 1),
                  _row_spec(r, 64), _full_spec((1, 64)), _full_spec((1, 64)),
                  _full_spec((1, 64)), _full_spec((64, 32)),
                  _full_spec((1, 32)), _full_spec((1, 32)),
                  _full_spec((1, 32)), _full_spec((32, 2)),
                  _full_spec((1, 2))],
        out_specs=[_row_spec(r, 2)],
        out_shape=[jax.ShapeDtypeStruct((n, 2), jnp.float32)],
        name="tc_gnn_final",
    )(acc_lo, acc_hi, dinv, hprev, b, g, bt, Wf1, bf1, gf1, btf1, Wf2, bf2)[0]


def kernel(x, edge_index, W_enc, b_enc, W1, b1, g1, bt1, W2, b2, g2, bt2,
           W3, b3, g3, bt3, Wf1, bf1, gf1, btf1, Wf2, bf2):
    n = x.shape[0]
    e = edge_index.shape[1]
    r = 1000 if n % 1000 == 0 else 8
    np_ = -(-n // (NS * 8)) * (NS * 8)
    rps = np_ // NS

    per_w = -(-e // (CH * NC * NS))
    e_pad = per_w * CH * NC * NS
    src = jnp.concatenate([edge_index[0], jnp.zeros((e_pad - e,), jnp.int32)])
    dst = jnp.concatenate([edge_index[1],
                           jnp.full((e_pad - e,), n, jnp.int32)])
    cps = e_pad // (CH * NS)

    pdeg = _sc_degree(dst, np_=np_, rps=rps, cpw=per_w)
    p0, p1 = pdeg[:n], pdeg[np_:np_ + n]

    be = b_enc.reshape(1, 64)
    h0, lo, hi, dinv = _tc_pre(x, p0, p1, W_enc, be, W1, n=n, np_=np_, r=r)

    agg = functools.partial(_sc_aggregate, src=src, dst=dst,
                            np_=np_, rps=rps, cps=cps)

    a_lo, a_hi = agg(lo, hi)
    h1, lo, hi = _tc_mid(a_lo, a_hi, dinv, h0, b1.reshape(1, 64),
                         g1.reshape(1, 64), bt1.reshape(1, 64), W2,
                         n=n, np_=np_, r=r)
    a_lo, a_hi = agg(lo, hi)
    h2, lo, hi = _tc_mid(a_lo, a_hi, dinv, h1, b2.reshape(1, 64),
                         g2.reshape(1, 64), bt2.reshape(1, 64), W3,
                         n=n, np_=np_, r=r)
    a_lo, a_hi = agg(lo, hi)
    return _tc_final(a_lo, a_hi, dinv, h2, b3.reshape(1, 64),
                     g3.reshape(1, 64), bt3.reshape(1, 64), Wf1,
                     bf1.reshape(1, 32), gf1.reshape(1, 32),
                     btf1.reshape(1, 32), Wf2, bf2.reshape(1, 2), n=n, r=r)


# speedup vs baseline: 17.1083x; 1.7106x over previous
import functools
import math

import jax
import jax.numpy as jnp
from jax import lax
from jax.experimental import pallas as pl
from jax.experimental.pallas import tpu as pltpu
from jax.experimental.pallas import tpu_sc as plsc

NC = 2
NS = 16
CH = 128
BN_SCALE = 1.0 / math.sqrt(1.0 + 1e-5)


DEG_NB = 3
AGG_NB = 3


def _deg_body(np_, rps, cpw, dst_ref, out_ref, acc, idxd, ones, zbuf, semi,
              sems):
    nb = DEG_NB
    c = lax.axis_index("c")
    s = lax.axis_index("s")
    w = c * NS + s
    base = w * cpw
    grp = cpw // nb

    def fill_z(i, _):
        zbuf[i, :] = jnp.zeros((16,), jnp.float32)
        return 0

    lax.fori_loop(0, rps, fill_z, 0)

    def fill_o(i, _):
        ones[i, :] = jnp.ones((16,), jnp.float32)
        return 0

    lax.fori_loop(0, CH, fill_o, 0)

    pltpu.sync_copy(zbuf, acc.at[pl.ds(s * rps, rps)])
    plsc.subcore_barrier()

    def idx_issue(g, po):
        for b in range(nb):
            ch = base + g * nb + b
            pltpu.async_copy(dst_ref.at[pl.ds(ch * CH, CH)], idxd.at[po + b],
                             semi)
        for b in range(nb):
            pltpu.make_async_copy(dst_ref.at[pl.ds(0, CH)], idxd.at[po + b],
                                  semi).wait()

    def scatter_issue(po):
        for b in range(nb):
            pltpu.async_copy(ones, acc.at[idxd.at[po + b]], sems, add=True)

    def scatter_wait(po):
        for b in range(nb):
            pltpu.make_async_copy(ones, acc.at[pl.ds(0, CH)], sems).wait()

    idx_issue(0, 0)

    def loop_body(t, _):
        po = (t % 2) * nb
        qo = nb - po

        @pl.when(t > 0)
        def _():
            scatter_wait(qo)

        scatter_issue(po)
        idx_issue(jnp.minimum(t + 1, grp - 1), qo)
        return 0

    lax.fori_loop(0, grp, loop_body, 0)
    scatter_wait(((grp - 1) % 2) * nb)
    plsc.subcore_barrier()
    pltpu.sync_copy(acc.at[pl.ds(s * rps, rps)],
                    out_ref.at[pl.ds(c * np_ + s * rps, rps)])


def _scatter_body(rps, cps, lo_ref, hi_ref, src_ref, dst_ref,
                  out_lo_ref, out_hi_ref, acc, idxs, idxd, rows,
                  semi, semg, sems):
    nb = AGG_NB
    c = lax.axis_index("c")
    s = lax.axis_index("s")
    grp = cps // nb

    def run(hs_ref, out_ref):
        base = s * cps
        pltpu.sync_copy(hs_ref.at[pl.ds(s * rps, rps)],
                        acc.at[pl.ds(s * rps, rps)])
        plsc.subcore_barrier()

        def idx_issue(g, po):
            for b in range(nb):
                ch = base + g * nb + b
                pltpu.async_copy(src_ref.at[pl.ds(ch * CH, CH)],
                                 idxs.at[po + b], semi)
                pltpu.async_copy(dst_ref.at[pl.ds(ch * CH, CH)],
                                 idxd.at[po + b], semi)
            for b in range(nb):
                pltpu.make_async_copy(src_ref.at[pl.ds(0, CH)],
                                      idxs.at[po + b], semi).wait()
                pltpu.make_async_copy(src_ref.at[pl.ds(0, CH)],
                                      idxd.at[po + b], semi).wait()

        def gather_issue(po):
            for b in range(nb):
                pltpu.async_copy(hs_ref.at[idxs.at[po + b]],
                                 rows.at[pl.ds((po + b) * CH, CH)], semg)

        def gather_wait(po):
            for b in range(nb):
                pltpu.make_async_copy(hs_ref.at[pl.ds(0, CH)],
                                      rows.at[pl.ds((po + b) * CH, CH)],
                                      semg).wait()

        def scatter_issue(po):
            for b in range(nb):
                pltpu.async_copy(rows.at[pl.ds((po + b) * CH, CH)],
                                 acc.at[idxd.at[po + b]], sems, add=True)

        def scatter_wait(po):
            for b in range(nb):
                pltpu.make_async_copy(rows.at[pl.ds((po + b) * CH, CH)],
                                      acc.at[pl.ds(0, CH)], sems).wait()

        idx_issue(0, 0)
        gather_issue(0)

        def loop_body(t, _):
            po = (t % 2) * nb
            qo = nb - po
            gather_wait(po)

            @pl.when(t > 0)
            def _():
                scatter_wait(qo)

            scatter_issue(po)
            idx_issue(jnp.minimum(t + 1, grp - 1), qo)
            gather_issue(qo)
            return 0

        lax.fori_loop(0, grp, loop_body, 0)
        gather_wait((grp % 2) * nb)
        scatter_wait(((grp - 1) % 2) * nb)
        plsc.subcore_barrier()
        pltpu.sync_copy(acc.at[pl.ds(s * rps, rps)],
                        out_ref.at[pl.ds(s * rps, rps)])

    pl.when(c == 0)(lambda: run(lo_ref, out_lo_ref))
    pl.when(c == 1)(lambda: run(hi_ref, out_hi_ref))


@functools.partial(jax.jit, static_argnames=("np_", "rps", "cpw"))
def _sc_degree(dst, *, np_, rps, cpw):
    mesh = plsc.VectorSubcoreMesh(core_axis_name="c", subcore_axis_name="s")
    body = functools.partial(_deg_body, np_, rps, cpw)
    return pl.kernel(
        body,
        out_type=jax.ShapeDtypeStruct((NC * np_, 16), jnp.float32),
        mesh=mesh,
        scratch_types=[
            pltpu.VMEM_SHARED((np_, 16), jnp.float32),
            pltpu.VMEM((2 * DEG_NB, CH), jnp.int32),
            pltpu.VMEM((CH, 16), jnp.float32),
            pltpu.VMEM((rps, 16), jnp.float32),
            pltpu.SemaphoreType.DMA,
            pltpu.SemaphoreType.DMA,
        ],
        compiler_params=pltpu.CompilerParams(use_tc_tiling_on_sc=False),
        name="sc_gcn_degree",
    )(dst)


@functools.partial(jax.jit, static_argnames=("np_", "rps", "cps"))
def _sc_aggregate(hs_lo, hs_hi, src, dst, *, np_, rps, cps):
    mesh = plsc.VectorSubcoreMesh(core_axis_name="c", subcore_axis_name="s")
    body = functools.partial(_scatter_body, rps, cps)
    return pl.kernel(
        body,
        out_type=(jax.ShapeDtypeStruct((np_, 32), jnp.float32),
                  jax.ShapeDtypeStruct((np_, 32), jnp.float32)),
        mesh=mesh,
        scratch_types=[
            pltpu.VMEM_SHARED((np_, 32), jnp.float32),
            pltpu.VMEM((2 * AGG_NB, CH), jnp.int32),
            pltpu.VMEM((2 * AGG_NB, CH), jnp.int32),
            pltpu.VMEM((2 * AGG_NB * CH, 32), jnp.float32),
            pltpu.SemaphoreType.DMA,
            pltpu.SemaphoreType.DMA,
            pltpu.SemaphoreType.DMA,
        ],
        compiler_params=pltpu.CompilerParams(use_tc_tiling_on_sc=False),
        name="sc_gcn_aggregate",
    )(hs_lo, hs_hi, src, dst)


def _pre_body(x_ref, p0_ref, p1_ref, We_ref, be_ref, W1_ref,
              h0_ref, lo_ref, hi_ref, dinv_ref):
    deg = 1.0 + p0_ref[:, :1] + p1_ref[:, :1]
    dinv = lax.rsqrt(deg)
    h0 = jax.nn.relu(jnp.dot(x_ref[:], We_ref[:],
                             preferred_element_type=jnp.float32) + be_ref[:])
    hs = dinv * jnp.dot(h0, W1_ref[:], preferred_element_type=jnp.float32)
    h0_ref[:] = h0
    lo_ref[:] = hs[:, :32]
    hi_ref[:] = hs[:, 32:]
    dinv_ref[:] = dinv


def _mid_body(lo_ref, hi_ref, dinv_ref, hprev_ref, b_ref, g_ref, bt_ref,
              Wn_ref, h_ref, nlo_ref, nhi_ref):
    accf = jnp.concatenate([lo_ref[:], hi_ref[:]], axis=1)
    dinv = dinv_ref[:]
    gcn = dinv * accf + b_ref[:]
    t = jax.nn.relu(g_ref[:] * (gcn * BN_SCALE) + bt_ref[:]) + hprev_ref[:]
    hs = dinv * jnp.dot(t, Wn_ref[:], preferred_element_type=jnp.float32)
    h_ref[:] = t
    nlo_ref[:] = hs[:, :32]
    nhi_ref[:] = hs[:, 32:]


def _final_body(lo_ref, hi_ref, dinv_ref, hprev_ref, b_ref, g_ref, bt_ref,
                Wf1_ref, bf1_ref, gf1_ref, btf1_ref, Wf2_ref, bf2_ref,
                out_ref):
    accf = jnp.concatenate([lo_ref[:], hi_ref[:]], axis=1)
    gcn = dinv_ref[:] * accf + b_ref[:]
    t = jax.nn.relu(g_ref[:] * (gcn * BN_SCALE) + bt_ref[:]) + hprev_ref[:]
    z = jnp.dot(t, Wf1_ref[:], preferred_element_type=jnp.float32) + bf1_ref[:]
    z = jax.nn.relu(gf1_ref[:] * (z * BN_SCALE) + btf1_ref[:])
    out_ref[:] = jnp.tanh(
        jnp.dot(z, Wf2_ref[:], preferred_element_type=jnp.float32) + bf2_ref[:])


def _row_spec(r, cols):
    return pl.BlockSpec((r, cols), lambda i: (i, 0))


def _full_spec(shape):
    return pl.BlockSpec(shape, lambda i: tuple(0 for _ in shape))


def _tc_pre(x, p0, p1, We, be, W1, *, n, np_, r):
    grid = (n // r,)
    return pl.pallas_call(
        _pre_body,
        grid=grid,
        in_specs=[_row_spec(r, 2), _row_spec(r, 16), _row_spec(r, 16),
                  _full_spec((2, 64)), _full_spec((1, 64)),
                  _full_spec((64, 64))],
        out_specs=[_row_spec(r, 64), _row_spec(r, 32), _row_spec(r, 32),
                   _row_spec(r, 1)],
        out_shape=[jax.ShapeDtypeStruct((n, 64), jnp.float32),
                   jax.ShapeDtypeStruct((np_, 32), jnp.float32),
                   jax.ShapeDtypeStruct((np_, 32), jnp.float32),
                   jax.ShapeDtypeStruct((n, 1), jnp.float32)],
        name="tc_gnn_pre",
    )(x, p0, p1, We, be, W1)


def _tc_mid(acc_lo, acc_hi, dinv, hprev, b, g, bt, Wn, *, n, np_, r):
    grid = (n // r,)
    return pl.pallas_call(
        _mid_body,
        grid=grid,
        in_specs=[_row_spec(r, 32), _row_spec(r, 32), _row_spec(r, 1),
                  _row_spec(r, 64), _full_spec((1, 64)), _full_spec((1, 64)),
                  _full_spec((1, 64)), _full_spec((64, 64))],
        out_specs=[_row_spec(r, 64), _row_spec(r, 32), _row_spec(r, 32)],
        out_shape=[jax.ShapeDtypeStruct((n, 64), jnp.float32),
                   jax.ShapeDtypeStruct((np_, 32), jnp.float32),
                   jax.ShapeDtypeStruct((np_, 32), jnp.float32)],
        name="tc_gnn_mid",
    )(acc_lo, acc_hi, dinv, hprev, b, g, bt, Wn)


def _tc_final(acc_lo, acc_hi, dinv, hprev, b, g, bt, Wf1, bf1, gf1, btf1,
              Wf2, bf2, *, n, r):
    grid = (n // r,)
    return pl.pallas_call(
        _final_body,
        grid=grid,
        in_specs=[_row_spec(r, 32), _row_spec(r, 32), _row_spec(r, 1),
                  _row_spec(r, 64), _full_spec((1, 64)), _full_spec((1, 64)),
                  _full_spec((1, 64)), _full_spec((64, 32)),
                  _full_spec((1, 32)), _full_spec((1, 32)),
                  _full_spec((1, 32)), _full_spec((32, 2)),
                  _full_spec((1, 2))],
        out_specs=[_row_spec(r, 2)],
        out_shape=[jax.ShapeDtypeStruct((n, 2), jnp.float32)],
        name="tc_gnn_final",
    )(acc_lo, acc_hi, dinv, hprev, b, g, bt, Wf1, bf1, gf1, btf1, Wf2, bf2)[0]


def kernel(x, edge_index, W_enc, b_enc, W1, b1, g1, bt1, W2, b2, g2, bt2,
           W3, b3, g3, bt3, Wf1, bf1, gf1, btf1, Wf2, bf2):
    n = x.shape[0]
    e = edge_index.shape[1]
    r = 1000 if n % 1000 == 0 else 8
    np_ = -(-n // (NS * 8)) * (NS * 8)
    rps = np_ // NS

    per_w = -(-e // (CH * NC * NS * DEG_NB)) * DEG_NB
    e_pad = per_w * CH * NC * NS
    src = jnp.concatenate([edge_index[0], jnp.zeros((e_pad - e,), jnp.int32)])
    dst = jnp.concatenate([edge_index[1],
                           jnp.full((e_pad - e,), n, jnp.int32)])
    cps = e_pad // (CH * NS)

    pdeg = _sc_degree(dst, np_=np_, rps=rps, cpw=per_w)
    p0, p1 = pdeg[:n], pdeg[np_:np_ + n]

    be = b_enc.reshape(1, 64)
    h0, lo, hi, dinv = _tc_pre(x, p0, p1, W_enc, be, W1, n=n, np_=np_, r=r)

    agg = functools.partial(_sc_aggregate, src=src, dst=dst,
                            np_=np_, rps=rps, cps=cps)

    a_lo, a_hi = agg(lo, hi)
    h1, lo, hi = _tc_mid(a_lo, a_hi, dinv, h0, b1.reshape(1, 64),
                         g1.reshape(1, 64), bt1.reshape(1, 64), W2,
                         n=n, np_=np_, r=r)
    a_lo, a_hi = agg(lo, hi)
    h2, lo, hi = _tc_mid(a_lo, a_hi, dinv, h1, b2.reshape(1, 64),
                         g2.reshape(1, 64), bt2.reshape(1, 64), W3,
                         n=n, np_=np_, r=r)
    a_lo, a_hi = agg(lo, hi)
    return _tc_final(a_lo, a_hi, dinv, h2, b3.reshape(1, 64),
                     g3.reshape(1, 64), bt3.reshape(1, 64), Wf1,
                     bf1.reshape(1, 32), gf1.reshape(1, 32),
                     btf1.reshape(1, 32), Wf2, bf2.reshape(1, 2), n=n, r=r)


# speedup vs baseline: 19.0248x; 1.1120x over previous
import functools
import math

import jax
import jax.numpy as jnp
from jax import lax
from jax.experimental import pallas as pl
from jax.experimental.pallas import tpu as pltpu
from jax.experimental.pallas import tpu_sc as plsc

NC = 2
NS = 16
CH = 128
BN_SCALE = 1.0 / math.sqrt(1.0 + 1e-5)


DEG_NB = 3
AGG_NB = 3


def _deg_body(np_, rps, cpw, dst_ref, out_ref, acc, idxd, ones, zbuf, semi,
              sems):
    nb = DEG_NB
    c = lax.axis_index("c")
    s = lax.axis_index("s")
    w = c * NS + s
    base = w * cpw
    grp = cpw // nb

    def fill_z(i, _):
        zbuf[i, :] = jnp.zeros((16,), jnp.float32)
        return 0

    lax.fori_loop(0, rps, fill_z, 0)

    def fill_o(i, _):
        ones[i, :] = jnp.ones((16,), jnp.float32)
        return 0

    lax.fori_loop(0, CH, fill_o, 0)

    pltpu.sync_copy(zbuf, acc.at[pl.ds(s * rps, rps)])
    plsc.subcore_barrier()

    def idx_issue(g, po):
        for b in range(nb):
            ch = base + g * nb + b
            pltpu.async_copy(dst_ref.at[pl.ds(ch * CH, CH)], idxd.at[po + b],
                             semi)
        for b in range(nb):
            pltpu.make_async_copy(dst_ref.at[pl.ds(0, CH)], idxd.at[po + b],
                                  semi).wait()

    def scatter_issue(po):
        for b in range(nb):
            pltpu.async_copy(ones, acc.at[idxd.at[po + b]], sems, add=True)

    def scatter_wait(po):
        for b in range(nb):
            pltpu.make_async_copy(ones, acc.at[pl.ds(0, CH)], sems).wait()

    idx_issue(0, 0)

    def loop_body(t, _):
        po = (t % 2) * nb
        qo = nb - po

        @pl.when(t > 0)
        def _():
            scatter_wait(qo)

        scatter_issue(po)
        idx_issue(jnp.minimum(t + 1, grp - 1), qo)
        return 0

    lax.fori_loop(0, grp, loop_body, 0)
    scatter_wait(((grp - 1) % 2) * nb)
    plsc.subcore_barrier()
    pltpu.sync_copy(acc.at[pl.ds(s * rps, rps)],
                    out_ref.at[pl.ds(c * np_ + s * rps, rps)])


def _scatter_body(rps, cps, lo_ref, hi_ref, src_ref, dst_ref,
                  out_lo_ref, out_hi_ref, acc, idxs, idxd, rows,
                  semi, semg, sems):
    nb = AGG_NB
    c = lax.axis_index("c")
    s = lax.axis_index("s")
    grp = cps // nb

    def run(hs_ref, out_ref):
        base = s * cps
        pltpu.sync_copy(hs_ref.at[pl.ds(s * rps, rps)],
                        acc.at[pl.ds(s * rps, rps)])
        plsc.subcore_barrier()

        def idx_issue(g, so):
            for b in range(nb):
                ch = base + g * nb + b
                pltpu.async_copy(src_ref.at[pl.ds(ch * CH, CH)],
                                 idxs.at[so + b], semi)
                pltpu.async_copy(dst_ref.at[pl.ds(ch * CH, CH)],
                                 idxd.at[so + b], semi)

        def idx_wait(so):
            for b in range(nb):
                pltpu.make_async_copy(src_ref.at[pl.ds(0, CH)],
                                      idxs.at[so + b], semi).wait()
                pltpu.make_async_copy(src_ref.at[pl.ds(0, CH)],
                                      idxd.at[so + b], semi).wait()

        def gather_issue(po, so):
            for b in range(nb):
                pltpu.async_copy(hs_ref.at[idxs.at[so + b]],
                                 rows.at[pl.ds((po + b) * CH, CH)], semg)

        def gather_wait(po):
            for b in range(nb):
                pltpu.make_async_copy(hs_ref.at[pl.ds(0, CH)],
                                      rows.at[pl.ds((po + b) * CH, CH)],
                                      semg).wait()

        def scatter_issue(po, so):
            for b in range(nb):
                pltpu.async_copy(rows.at[pl.ds((po + b) * CH, CH)],
                                 acc.at[idxd.at[so + b]], sems, add=True)

        def scatter_wait(po):
            for b in range(nb):
                pltpu.make_async_copy(rows.at[pl.ds((po + b) * CH, CH)],
                                      acc.at[pl.ds(0, CH)], sems).wait()

        idx_issue(0, 0)
        idx_wait(0)
        gather_issue(0, 0)
        idx_issue(jnp.minimum(1, grp - 1), nb)

        def loop_body(t, _):
            po = (t % 2) * nb
            qo = nb - po
            so = (t % 3) * nb
            so1 = ((t + 1) % 3) * nb
            so2 = ((t + 2) % 3) * nb
            gather_wait(po)

            @pl.when(t > 0)
            def _():
                scatter_wait(qo)

            scatter_issue(po, so)
            idx_wait(so1)
            gather_issue(qo, so1)
            idx_issue(jnp.minimum(t + 2, grp - 1), so2)
            return 0

        lax.fori_loop(0, grp, loop_body, 0)
        gather_wait((grp % 2) * nb)
        scatter_wait(((grp - 1) % 2) * nb)
        idx_wait(((grp + 1) % 3) * nb)
        plsc.subcore_barrier()
        pltpu.sync_copy(acc.at[pl.ds(s * rps, rps)],
                        out_ref.at[pl.ds(s * rps, rps)])

    pl.when(c == 0)(lambda: run(lo_ref, out_lo_ref))
    pl.when(c == 1)(lambda: run(hi_ref, out_hi_ref))


@functools.partial(jax.jit, static_argnames=("np_", "rps", "cpw"))
def _sc_degree(dst, *, np_, rps, cpw):
    mesh = plsc.VectorSubcoreMesh(core_axis_name="c", subcore_axis_name="s")
    body = functools.partial(_deg_body, np_, rps, cpw)
    return pl.kernel(
        body,
        out_type=jax.ShapeDtypeStruct((NC * np_, 16), jnp.float32),
        mesh=mesh,
        scratch_types=[
            pltpu.VMEM_SHARED((np_, 16), jnp.float32),
            pltpu.VMEM((2 * DEG_NB, CH), jnp.int32),
            pltpu.VMEM((CH, 16), jnp.float32),
            pltpu.VMEM((rps, 16), jnp.float32),
            pltpu.SemaphoreType.DMA,
            pltpu.SemaphoreType.DMA,
        ],
        compiler_params=pltpu.CompilerParams(use_tc_tiling_on_sc=False),
        name="sc_gcn_degree",
    )(dst)


@functools.partial(jax.jit, static_argnames=("np_", "rps", "cps"))
def _sc_aggregate(hs_lo, hs_hi, src, dst, *, np_, rps, cps):
    mesh = plsc.VectorSubcoreMesh(core_axis_name="c", subcore_axis_name="s")
    body = functools.partial(_scatter_body, rps, cps)
    return pl.kernel(
        body,
        out_type=(jax.ShapeDtypeStruct((np_, 32), jnp.float32),
                  jax.ShapeDtypeStruct((np_, 32), jnp.float32)),
        mesh=mesh,
        scratch_types=[
            pltpu.VMEM_SHARED((np_, 32), jnp.float32),
            pltpu.VMEM((3 * AGG_NB, CH), jnp.int32),
            pltpu.VMEM((3 * AGG_NB, CH), jnp.int32),
            pltpu.VMEM((2 * AGG_NB * CH, 32), jnp.float32),
            pltpu.SemaphoreType.DMA,
            pltpu.SemaphoreType.DMA,
            pltpu.SemaphoreType.DMA,
        ],
        compiler_params=pltpu.CompilerParams(use_tc_tiling_on_sc=False),
        name="sc_gcn_aggregate",
    )(hs_lo, hs_hi, src, dst)


def _pre_body(x_ref, p0_ref, p1_ref, We_ref, be_ref, W1_ref,
              h0_ref, lo_ref, hi_ref, dinv_ref):
    deg = 1.0 + p0_ref[:, :1] + p1_ref[:, :1]
    dinv = lax.rsqrt(deg)
    h0 = jax.nn.relu(jnp.dot(x_ref[:], We_ref[:],
                             preferred_element_type=jnp.float32) + be_ref[:])
    hs = dinv * jnp.dot(h0, W1_ref[:], preferred_element_type=jnp.float32)
    h0_ref[:] = h0
    lo_ref[:] = hs[:, :32]
    hi_ref[:] = hs[:, 32:]
    dinv_ref[:] = dinv


def _mid_body(lo_ref, hi_ref, dinv_ref, hprev_ref, b_ref, g_ref, bt_ref,
              Wn_ref, h_ref, nlo_ref, nhi_ref):
    accf = jnp.concatenate([lo_ref[:], hi_ref[:]], axis=1)
    dinv = dinv_ref[:]
    gcn = dinv * accf + b_ref[:]
    t = jax.nn.relu(g_ref[:] * (gcn * BN_SCALE) + bt_ref[:]) + hprev_ref[:]
    hs = dinv * jnp.dot(t, Wn_ref[:], preferred_element_type=jnp.float32)
    h_ref[:] = t
    nlo_ref[:] = hs[:, :32]
    nhi_ref[:] = hs[:, 32:]


def _final_body(lo_ref, hi_ref, dinv_ref, hprev_ref, b_ref, g_ref, bt_ref,
                Wf1_ref, bf1_ref, gf1_ref, btf1_ref, Wf2_ref, bf2_ref,
                out_ref):
    accf = jnp.concatenate([lo_ref[:], hi_ref[:]], axis=1)
    gcn = dinv_ref[:] * accf + b_ref[:]
    t = jax.nn.relu(g_ref[:] * (gcn * BN_SCALE) + bt_ref[:]) + hprev_ref[:]
    z = jnp.dot(t, Wf1_ref[:], preferred_element_type=jnp.float32) + bf1_ref[:]
    z = jax.nn.relu(gf1_ref[:] * (z * BN_SCALE) + btf1_ref[:])
    out_ref[:] = jnp.tanh(
        jnp.dot(z, Wf2_ref[:], preferred_element_type=jnp.float32) + bf2_ref[:])


def _row_spec(r, cols):
    return pl.BlockSpec((r, cols), lambda i: (i, 0))


def _full_spec(shape):
    return pl.BlockSpec(shape, lambda i: tuple(0 for _ in shape))


def _tc_pre(x, p0, p1, We, be, W1, *, n, np_, r):
    grid = (n // r,)
    return pl.pallas_call(
        _pre_body,
        grid=grid,
        in_specs=[_row_spec(r, 2), _row_spec(r, 16), _row_spec(r, 16),
                  _full_spec((2, 64)), _full_spec((1, 64)),
                  _full_spec((64, 64))],
        out_specs=[_row_spec(r, 64), _row_spec(r, 32), _row_spec(r, 32),
                   _row_spec(r, 1)],
        out_shape=[jax.ShapeDtypeStruct((n, 64), jnp.float32),
                   jax.ShapeDtypeStruct((np_, 32), jnp.float32),
                   jax.ShapeDtypeStruct((np_, 32), jnp.float32),
                   jax.ShapeDtypeStruct((n, 1), jnp.float32)],
        name="tc_gnn_pre",
    )(x, p0, p1, We, be, W1)


def _tc_mid(acc_lo, acc_hi, dinv, hprev, b, g, bt, Wn, *, n, np_, r):
    grid = (n // r,)
    return pl.pallas_call(
        _mid_body,
        grid=grid,
        in_specs=[_row_spec(r, 32), _row_spec(r, 32), _row_spec(r, 1),
                  _row_spec(r, 64), _full_spec((1, 64)), _full_spec((1, 64)),
                  _full_spec((1, 64)), _full_spec((64, 64))],
        out_specs=[_row_spec(r, 64), _row_spec(r, 32), _row_spec(r, 32)],
        out_shape=[jax.ShapeDtypeStruct((n, 64), jnp.float32),
                   jax.ShapeDtypeStruct((np_, 32), jnp.float32),
                   jax.ShapeDtypeStruct((np_, 32), jnp.float32)],
        name="tc_gnn_mid",
    )(acc_lo, acc_hi, dinv, hprev, b, g, bt, Wn)


def _tc_final(acc_lo, acc_hi, dinv, hprev, b, g, bt, Wf1, bf1, gf1, btf1,
              Wf2, bf2, *, n, r):
    grid = (n // r,)
    return pl.pallas_call(
        _final_body,
        grid=grid,
        in_specs=[_row_spec(r, 32), _row_spec(r, 32), _row_spec(r, 1),
                  _row_spec(r, 64), _full_spec((1, 64)), _full_spec((1, 64)),
                  _full_spec((1, 64)), _full_spec((64, 32)),
                  _full_spec((1, 32)), _full_spec((1, 32)),
                  _full_spec((1, 32)), _full_spec((32, 2)),
                  _full_spec((1, 2))],
        out_specs=[_row_spec(r, 2)],
        out_shape=[jax.ShapeDtypeStruct((n, 2), jnp.float32)],
        name="tc_gnn_final",
    )(acc_lo, acc_hi, dinv, hprev, b, g, bt, Wf1, bf1, gf1, btf1, Wf2, bf2)[0]


def kernel(x, edge_index, W_enc, b_enc, W1, b1, g1, bt1, W2, b2, g2, bt2,
           W3, b3, g3, bt3, Wf1, bf1, gf1, btf1, Wf2, bf2):
    n = x.shape[0]
    e = edge_index.shape[1]
    r = 1000 if n % 1000 == 0 else 8
    np_ = -(-n // (NS * 8)) * (NS * 8)
    rps = np_ // NS

    per_w = -(-e // (CH * NC * NS * DEG_NB)) * DEG_NB
    e_pad = per_w * CH * NC * NS
    src = jnp.concatenate([edge_index[0], jnp.zeros((e_pad - e,), jnp.int32)])
    dst = jnp.concatenate([edge_index[1],
                           jnp.full((e_pad - e,), n, jnp.int32)])
    cps = e_pad // (CH * NS)

    pdeg = _sc_degree(dst, np_=np_, rps=rps, cpw=per_w)
    p0, p1 = pdeg[:n], pdeg[np_:np_ + n]

    be = b_enc.reshape(1, 64)
    h0, lo, hi, dinv = _tc_pre(x, p0, p1, W_enc, be, W1, n=n, np_=np_, r=r)

    agg = functools.partial(_sc_aggregate, src=src, dst=dst,
                            np_=np_, rps=rps, cps=cps)

    a_lo, a_hi = agg(lo, hi)
    h1, lo, hi = _tc_mid(a_lo, a_hi, dinv, h0, b1.reshape(1, 64),
                         g1.reshape(1, 64), bt1.reshape(1, 64), W2,
                         n=n, np_=np_, r=r)
    a_lo, a_hi = agg(lo, hi)
    h2, lo, hi = _tc_mid(a_lo, a_hi, dinv, h1, b2.reshape(1, 64),
                         g2.reshape(1, 64), bt2.reshape(1, 64), W3,
                         n=n, np_=np_, r=r)
    a_lo, a_hi = agg(lo, hi)
    return _tc_final(a_lo, a_hi, dinv, h2, b3.reshape(1, 64),
                     g3.reshape(1, 64), bt3.reshape(1, 64), Wf1,
                     bf1.reshape(1, 32), gf1.reshape(1, 32),
                     btf1.reshape(1, 32), Wf2, bf2.reshape(1, 2), n=n, r=r)


# speedup vs baseline: 19.8958x; 1.0458x over previous
import functools
import math

import jax
import jax.numpy as jnp
from jax import lax
from jax.experimental import pallas as pl
from jax.experimental.pallas import tpu as pltpu
from jax.experimental.pallas import tpu_sc as plsc

NC = 2
NS = 16
CH = 128
BN_SCALE = 1.0 / math.sqrt(1.0 + 1e-5)


DEG_NB = 3
AGG_NB = 3


def _deg_body(np_, rps, cpw, dst_ref, out_ref, acc, idxd, ones, zbuf, semi,
              sems):
    nb = DEG_NB
    c = lax.axis_index("c")
    s = lax.axis_index("s")
    w = c * NS + s
    base = w * cpw
    grp = cpw // nb

    def fill_z(i, _):
        zbuf[i, :] = jnp.zeros((16,), jnp.float32)
        return 0

    lax.fori_loop(0, rps, fill_z, 0)

    def fill_o(i, _):
        ones[i, :] = jnp.ones((16,), jnp.float32)
        return 0

    lax.fori_loop(0, CH, fill_o, 0)

    pltpu.sync_copy(zbuf, acc.at[pl.ds(s * rps, rps)])
    plsc.subcore_barrier()

    def idx_issue(g, po):
        for b in range(nb):
            ch = base + g * nb + b
            pltpu.async_copy(dst_ref.at[pl.ds(ch * CH, CH)], idxd.at[po + b],
                             semi)
        for b in range(nb):
            pltpu.make_async_copy(dst_ref.at[pl.ds(0, CH)], idxd.at[po + b],
                                  semi).wait()

    def scatter_issue(po):
        for b in range(nb):
            pltpu.async_copy(ones, acc.at[idxd.at[po + b]], sems, add=True)

    def scatter_wait(po):
        for b in range(nb):
            pltpu.make_async_copy(ones, acc.at[pl.ds(0, CH)], sems).wait()

    idx_issue(0, 0)

    def loop_body(t, _):
        po = (t % 2) * nb
        qo = nb - po

        @pl.when(t > 0)
        def _():
            scatter_wait(qo)

        scatter_issue(po)
        idx_issue(jnp.minimum(t + 1, grp - 1), qo)
        return 0

    lax.fori_loop(0, grp, loop_body, 0)
    scatter_wait(((grp - 1) % 2) * nb)
    plsc.subcore_barrier()
    pltpu.sync_copy(acc.at[pl.ds(s * rps, rps)],
                    out_ref.at[pl.ds(c * np_ + s * rps, rps)])


def _scatter_body(rps, cps, lo_ref, hi_ref, src_ref, dst_ref,
                  out_lo_ref, out_hi_ref, acc, idxs, idxd, rows,
                  semi, semg, sems):
    nb = AGG_NB
    c = lax.axis_index("c")
    s = lax.axis_index("s")
    grp = cps // nb

    def run(hs_ref, out_ref):
        base = s * cps
        pltpu.sync_copy(hs_ref.at[pl.ds(s * rps, rps)],
                        acc.at[pl.ds(s * rps, rps)])
        plsc.subcore_barrier()

        def idx_issue(g, so):
            for b in range(nb):
                ch = base + g * nb + b
                pltpu.async_copy(src_ref.at[pl.ds(ch * CH, CH)],
                                 idxs.at[so + b], semi)
                pltpu.async_copy(dst_ref.at[pl.ds(ch * CH, CH)],
                                 idxd.at[so + b], semi)

        def idx_wait(so):
            for b in range(nb):
                pltpu.make_async_copy(src_ref.at[pl.ds(0, CH)],
                                      idxs.at[so + b], semi).wait()
                pltpu.make_async_copy(src_ref.at[pl.ds(0, CH)],
                                      idxd.at[so + b], semi).wait()

        def gather_issue(po, so):
            for b in range(nb):
                pltpu.async_copy(hs_ref.at[idxs.at[so + b]],
                                 rows.at[pl.ds((po + b) * CH, CH)], semg)

        def gather_wait(po):
            for b in range(nb):
                pltpu.make_async_copy(hs_ref.at[pl.ds(0, CH)],
                                      rows.at[pl.ds((po + b) * CH, CH)],
                                      semg).wait()

        def scatter_issue(po, so):
            for b in range(nb):
                pltpu.async_copy(rows.at[pl.ds((po + b) * CH, CH)],
                                 acc.at[idxd.at[so + b]], sems, add=True)

        def scatter_wait(po):
            for b in range(nb):
                pltpu.make_async_copy(rows.at[pl.ds((po + b) * CH, CH)],
                                      acc.at[pl.ds(0, CH)], sems).wait()

        idx_issue(0, 0)
        idx_wait(0)
        gather_issue(0, 0)
        idx_issue(jnp.minimum(1, grp - 1), nb)

        def loop_body(t, _):
            po = (t % 2) * nb
            qo = nb - po
            so = (t % 3) * nb
            so1 = ((t + 1) % 3) * nb
            so2 = ((t + 2) % 3) * nb
            gather_wait(po)

            @pl.when(t > 0)
            def _():
                scatter_wait(qo)

            scatter_issue(po, so)
            idx_wait(so1)
            gather_issue(qo, so1)
            idx_issue(jnp.minimum(t + 2, grp - 1), so2)
            return 0

        lax.fori_loop(0, grp, loop_body, 0)
        gather_wait((grp % 2) * nb)
        scatter_wait(((grp - 1) % 2) * nb)
        idx_wait(((grp + 1) % 3) * nb)
        plsc.subcore_barrier()
        pltpu.sync_copy(acc.at[pl.ds(s * rps, rps)],
                        out_ref.at[pl.ds(s * rps, rps)])

    pl.when(c == 0)(lambda: run(lo_ref, out_lo_ref))
    pl.when(c == 1)(lambda: run(hi_ref, out_hi_ref))


@functools.partial(jax.jit, static_argnames=("np_", "rps", "cpw"))
def _sc_degree(dst, *, np_, rps, cpw):
    mesh = plsc.VectorSubcoreMesh(core_axis_name="c", subcore_axis_name="s")
    body = functools.partial(_deg_body, np_, rps, cpw)
    return pl.kernel(
        body,
        out_type=jax.ShapeDtypeStruct((NC * np_, 16), jnp.float32),
        mesh=mesh,
        scratch_types=[
            pltpu.VMEM_SHARED((np_, 16), jnp.float32),
            pltpu.VMEM((2 * DEG_NB, CH), jnp.int32),
            pltpu.VMEM((CH, 16), jnp.float32),
            pltpu.VMEM((rps, 16), jnp.float32),
            pltpu.SemaphoreType.DMA,
            pltpu.SemaphoreType.DMA,
        ],
        compiler_params=pltpu.CompilerParams(use_tc_tiling_on_sc=False),
        name="sc_gcn_degree",
    )(dst)


@functools.partial(jax.jit, static_argnames=("np_", "rps", "cps"))
def _sc_aggregate(hs_lo, hs_hi, src, dst, *, np_, rps, cps):
    mesh = plsc.VectorSubcoreMesh(core_axis_name="c", subcore_axis_name="s")
    body = functools.partial(_scatter_body, rps, cps)
    return pl.kernel(
        body,
        out_type=(jax.ShapeDtypeStruct((np_, 32), jnp.float32),
                  jax.ShapeDtypeStruct((np_, 32), jnp.float32)),
        mesh=mesh,
        scratch_types=[
            pltpu.VMEM_SHARED((np_, 32), jnp.float32),
            pltpu.VMEM((3 * AGG_NB, CH), jnp.int32),
            pltpu.VMEM((3 * AGG_NB, CH), jnp.int32),
            pltpu.VMEM((2 * AGG_NB * CH, 32), jnp.float32),
            pltpu.SemaphoreType.DMA,
            pltpu.SemaphoreType.DMA,
            pltpu.SemaphoreType.DMA,
        ],
        compiler_params=pltpu.CompilerParams(use_tc_tiling_on_sc=False),
        name="sc_gcn_aggregate",
    )(hs_lo, hs_hi, src, dst)


def _pre_body(x_ref, p0_ref, p1_ref, We_ref, be_ref, W1_ref,
              h0_ref, lo_ref, hi_ref, dinv_ref):
    deg = 1.0 + p0_ref[:, :1] + p1_ref[:, :1]
    dinv = lax.rsqrt(deg)
    h0 = jax.nn.relu(jnp.dot(x_ref[:], We_ref[:],
                             preferred_element_type=jnp.float32) + be_ref[:])
    hs = dinv * jnp.dot(h0, W1_ref[:], preferred_element_type=jnp.float32)
    h0_ref[:] = h0
    lo_ref[:] = hs[:, :32]
    hi_ref[:] = hs[:, 32:]
    dinv_ref[:] = dinv


def _mid_body(lo_ref, hi_ref, dinv_ref, hprev_ref, b_ref, g_ref, bt_ref,
              Wn_ref, h_ref, nlo_ref, nhi_ref):
    accf = jnp.concatenate([lo_ref[:], hi_ref[:]], axis=1)
    dinv = dinv_ref[:]
    gcn = dinv * accf + b_ref[:]
    t = jax.nn.relu(g_ref[:] * (gcn * BN_SCALE) + bt_ref[:]) + hprev_ref[:]
    hs = dinv * jnp.dot(t, Wn_ref[:], preferred_element_type=jnp.float32)
    h_ref[:] = t
    nlo_ref[:] = hs[:, :32]
    nhi_ref[:] = hs[:, 32:]


def _final_body(lo_ref, hi_ref, dinv_ref, hprev_ref, b_ref, g_ref, bt_ref,
                Wf1_ref, bf1_ref, gf1_ref, btf1_ref, Wf2_ref, bf2_ref,
                out_ref):
    accf = jnp.concatenate([lo_ref[:], hi_ref[:]], axis=1)
    gcn = dinv_ref[:] * accf + b_ref[:]
    t = jax.nn.relu(g_ref[:] * (gcn * BN_SCALE) + bt_ref[:]) + hprev_ref[:]
    z = jnp.dot(t, Wf1_ref[:], preferred_element_type=jnp.float32) + bf1_ref[:]
    z = jax.nn.relu(gf1_ref[:] * (z * BN_SCALE) + btf1_ref[:])
    out_ref[:] = jnp.tanh(
        jnp.dot(z, Wf2_ref[:], preferred_element_type=jnp.float32) + bf2_ref[:])


def _row_spec(r, cols):
    return pl.BlockSpec((r, cols), lambda i: (i, 0))


def _full_spec(shape):
    return pl.BlockSpec(shape, lambda i: tuple(0 for _ in shape))


def _tc_pre(x, p0, p1, We, be, W1, *, n, np_, r):
    grid = (n // r,)
    return pl.pallas_call(
        _pre_body,
        grid=grid,
        in_specs=[_row_spec(r, 2), _row_spec(r, 16), _row_spec(r, 16),
                  _full_spec((2, 64)), _full_spec((1, 64)),
                  _full_spec((64, 64))],
        out_specs=[_row_spec(r, 64), _row_spec(r, 32), _row_spec(r, 32),
                   _row_spec(r, 1)],
        out_shape=[jax.ShapeDtypeStruct((n, 64), jnp.float32),
                   jax.ShapeDtypeStruct((np_, 32), jnp.float32),
                   jax.ShapeDtypeStruct((np_, 32), jnp.float32),
                   jax.ShapeDtypeStruct((n, 1), jnp.float32)],
        name="tc_gnn_pre",
    )(x, p0, p1, We, be, W1)


def _tc_mid(acc_lo, acc_hi, dinv, hprev, b, g, bt, Wn, *, n, np_, r):
    grid = (n // r,)
    return pl.pallas_call(
        _mid_body,
        grid=grid,
        in_specs=[_row_spec(r, 32), _row_spec(r, 32), _row_spec(r, 1),
                  _row_spec(r, 64), _full_spec((1, 64)), _full_spec((1, 64)),
                  _full_spec((1, 64)), _full_spec((64, 64))],
        out_specs=[_row_spec(r, 64), _row_spec(r, 32), _row_spec(r, 32)],
        out_shape=[jax.ShapeDtypeStruct((n, 64), jnp.float32),
                   jax.ShapeDtypeStruct((np_, 32), jnp.float32),
                   jax.ShapeDtypeStruct((np_, 32), jnp.float32)],
        name="tc_gnn_mid",
    )(acc_lo, acc_hi, dinv, hprev, b, g, bt, Wn)


def _tc_final(acc_lo, acc_hi, dinv, hprev, b, g, bt, Wf1, bf1, gf1, btf1,
              Wf2, bf2, *, n, r):
    grid = (n // r,)
    return pl.pallas_call(
        _final_body,
        grid=grid,
        in_specs=[_row_spec(r, 32), _row_spec(r, 32), _row_spec(r, 1),
                  _row_spec(r, 64), _full_spec((1, 64)), _full_spec((1, 64)),
                  _full_spec((1, 64)), _full_spec((64, 32)),
                  _full_spec((1, 32)), _full_spec((1, 32)),
                  _full_spec((1, 32)), _full_spec((32, 2)),
                  _full_spec((1, 2))],
        out_specs=[_row_spec(r, 2)],
        out_shape=[jax.ShapeDtypeStruct((n, 2), jnp.float32)],
        name="tc_gnn_final",
    )(acc_lo, acc_hi, dinv, hprev, b, g, bt, Wf1, bf1, gf1, btf1, Wf2, bf2)[0]


def kernel(x, edge_index, W_enc, b_enc, W1, b1, g1, bt1, W2, b2, g2, bt2,
           W3, b3, g3, bt3, Wf1, bf1, gf1, btf1, Wf2, bf2):
    n = x.shape[0]
    e = edge_index.shape[1]
    r = 5000 if n % 5000 == 0 else (1000 if n % 1000 == 0 else 8)
    np_ = -(-n // (NS * 8)) * (NS * 8)
    rps = np_ // NS

    per_w = -(-e // (CH * NC * NS * DEG_NB)) * DEG_NB
    e_pad = per_w * CH * NC * NS
    src = jnp.concatenate([edge_index[0], jnp.zeros((e_pad - e,), jnp.int32)])
    dst = jnp.concatenate([edge_index[1],
                           jnp.full((e_pad - e,), n, jnp.int32)])
    cps = e_pad // (CH * NS)

    pdeg = _sc_degree(dst, np_=np_, rps=rps, cpw=per_w)
    p0, p1 = pdeg[:n], pdeg[np_:np_ + n]

    be = b_enc.reshape(1, 64)
    h0, lo, hi, dinv = _tc_pre(x, p0, p1, W_enc, be, W1, n=n, np_=np_, r=r)

    agg = functools.partial(_sc_aggregate, src=src, dst=dst,
                            np_=np_, rps=rps, cps=cps)

    a_lo, a_hi = agg(lo, hi)
    h1, lo, hi = _tc_mid(a_lo, a_hi, dinv, h0, b1.reshape(1, 64),
                         g1.reshape(1, 64), bt1.reshape(1, 64), W2,
                         n=n, np_=np_, r=r)
    a_lo, a_hi = agg(lo, hi)
    h2, lo, hi = _tc_mid(a_lo, a_hi, dinv, h1, b2.reshape(1, 64),
                         g2.reshape(1, 64), bt2.reshape(1, 64), W3,
                         n=n, np_=np_, r=r)
    a_lo, a_hi = agg(lo, hi)
    return _tc_final(a_lo, a_hi, dinv, h2, b3.reshape(1, 64),
                     g3.reshape(1, 64), bt3.reshape(1, 64), Wf1,
                     bf1.reshape(1, 32), gf1.reshape(1, 32),
                     btf1.reshape(1, 32), Wf2, bf2.reshape(1, 2), n=n, r=r)


# speedup vs baseline: 20.3274x; 1.0217x over previous
import functools
import math

import jax
import jax.numpy as jnp
from jax import lax
from jax.experimental import pallas as pl
from jax.experimental.pallas import tpu as pltpu
from jax.experimental.pallas import tpu_sc as plsc

NC = 2
NS = 16
CH = 128
BN_SCALE = 1.0 / math.sqrt(1.0 + 1e-5)


DEG_NB = 3
AGG_NB = 3


def _deg_body(np_, rps, cpw, dst_ref, out_ref, acc, idxd, ones, zbuf, semi,
              sems):
    nb = DEG_NB
    c = lax.axis_index("c")
    s = lax.axis_index("s")
    w = c * NS + s
    base = w * cpw
    grp = cpw // nb
    zlen = -(-rps // 16) * 16

    def fill_z(i, _):
        zbuf[pl.ds(i * 16, 16)] = jnp.zeros((16,), jnp.float32)
        return 0

    lax.fori_loop(0, zlen // 16, fill_z, 0)

    def fill_o(i, _):
        ones[pl.ds(i * 16, 16)] = jnp.ones((16,), jnp.float32)
        return 0

    lax.fori_loop(0, CH // 16, fill_o, 0)

    pltpu.sync_copy(zbuf.at[pl.ds(0, rps)], acc.at[pl.ds(s * rps, rps)])
    plsc.subcore_barrier()

    def idx_issue(g, po):
        for b in range(nb):
            ch = base + g * nb + b
            pltpu.async_copy(dst_ref.at[pl.ds(ch * CH, CH)], idxd.at[po + b],
                             semi)
        for b in range(nb):
            pltpu.make_async_copy(dst_ref.at[pl.ds(0, CH)], idxd.at[po + b],
                                  semi).wait()

    def scatter_issue(po):
        for b in range(nb):
            pltpu.async_copy(ones, acc.at[idxd.at[po + b]], sems, add=True)

    def scatter_wait(po):
        for b in range(nb):
            pltpu.make_async_copy(ones, acc.at[pl.ds(0, CH)], sems).wait()

    idx_issue(0, 0)

    def loop_body(t, _):
        po = (t % 2) * nb
        qo = nb - po

        @pl.when(t > 0)
        def _():
            scatter_wait(qo)

        scatter_issue(po)
        idx_issue(jnp.minimum(t + 1, grp - 1), qo)
        return 0

    lax.fori_loop(0, grp, loop_body, 0)
    scatter_wait(((grp - 1) % 2) * nb)
    plsc.subcore_barrier()
    pltpu.sync_copy(acc.at[pl.ds(s * rps, rps)],
                    out_ref.at[pl.ds(c * np_ + s * rps, rps)])


def _scatter_body(rps, cps, lo_ref, hi_ref, src_ref, dst_ref,
                  out_lo_ref, out_hi_ref, acc, idxs, idxd, rows,
                  semi, semg, sems):
    nb = AGG_NB
    c = lax.axis_index("c")
    s = lax.axis_index("s")
    grp = cps // nb

    def run(hs_ref, out_ref):
        base = s * cps
        pltpu.sync_copy(hs_ref.at[pl.ds(s * rps, rps)],
                        acc.at[pl.ds(s * rps, rps)])
        plsc.subcore_barrier()

        def idx_issue(g, so):
            for b in range(nb):
                ch = base + g * nb + b
                pltpu.async_copy(src_ref.at[pl.ds(ch * CH, CH)],
                                 idxs.at[so + b], semi)
                pltpu.async_copy(dst_ref.at[pl.ds(ch * CH, CH)],
                                 idxd.at[so + b], semi)

        def idx_wait(so):
            for b in range(nb):
                pltpu.make_async_copy(src_ref.at[pl.ds(0, CH)],
                                      idxs.at[so + b], semi).wait()
                pltpu.make_async_copy(src_ref.at[pl.ds(0, CH)],
                                      idxd.at[so + b], semi).wait()

        def gather_issue(po, so):
            for b in range(nb):
                pltpu.async_copy(hs_ref.at[idxs.at[so + b]],
                                 rows.at[pl.ds((po + b) * CH, CH)], semg)

        def gather_wait(po):
            for b in range(nb):
                pltpu.make_async_copy(hs_ref.at[pl.ds(0, CH)],
                                      rows.at[pl.ds((po + b) * CH, CH)],
                                      semg).wait()

        def scatter_issue(po, so):
            for b in range(nb):
                pltpu.async_copy(rows.at[pl.ds((po + b) * CH, CH)],
                                 acc.at[idxd.at[so + b]], sems, add=True)

        def scatter_wait(po):
            for b in range(nb):
                pltpu.make_async_copy(rows.at[pl.ds((po + b) * CH, CH)],
                                      acc.at[pl.ds(0, CH)], sems).wait()

        idx_issue(0, 0)
        idx_wait(0)
        gather_issue(0, 0)
        idx_issue(jnp.minimum(1, grp - 1), nb)

        def loop_body(t, _):
            po = (t % 2) * nb
            qo = nb - po
            so = (t % 3) * nb
            so1 = ((t + 1) % 3) * nb
            so2 = ((t + 2) % 3) * nb
            gather_wait(po)

            @pl.when(t > 0)
            def _():
                scatter_wait(qo)

            scatter_issue(po, so)
            idx_wait(so1)
            gather_issue(qo, so1)
            idx_issue(jnp.minimum(t + 2, grp - 1), so2)
            return 0

        lax.fori_loop(0, grp, loop_body, 0)
        gather_wait((grp % 2) * nb)
        scatter_wait(((grp - 1) % 2) * nb)
        idx_wait(((grp + 1) % 3) * nb)
        plsc.subcore_barrier()
        pltpu.sync_copy(acc.at[pl.ds(s * rps, rps)],
                        out_ref.at[pl.ds(s * rps, rps)])

    pl.when(c == 0)(lambda: run(lo_ref, out_lo_ref))
    pl.when(c == 1)(lambda: run(hi_ref, out_hi_ref))


@functools.partial(jax.jit, static_argnames=("np_", "rps", "cpw"))
def _sc_degree(dst, *, np_, rps, cpw):
    mesh = plsc.VectorSubcoreMesh(core_axis_name="c", subcore_axis_name="s")
    body = functools.partial(_deg_body, np_, rps, cpw)
    return pl.kernel(
        body,
        out_type=jax.ShapeDtypeStruct((NC * np_,), jnp.float32),
        mesh=mesh,
        scratch_types=[
            pltpu.VMEM_SHARED((np_,), jnp.float32),
            pltpu.VMEM((2 * DEG_NB, CH), jnp.int32),
            pltpu.VMEM((CH,), jnp.float32),
            pltpu.VMEM((-(-rps // 16) * 16,), jnp.float32),
            pltpu.SemaphoreType.DMA,
            pltpu.SemaphoreType.DMA,
        ],
        compiler_params=pltpu.CompilerParams(use_tc_tiling_on_sc=False),
        name="sc_gcn_degree",
    )(dst)


@functools.partial(jax.jit, static_argnames=("np_", "rps", "cps"))
def _sc_aggregate(hs_lo, hs_hi, src, dst, *, np_, rps, cps):
    mesh = plsc.VectorSubcoreMesh(core_axis_name="c", subcore_axis_name="s")
    body = functools.partial(_scatter_body, rps, cps)
    return pl.kernel(
        body,
        out_type=(jax.ShapeDtypeStruct((np_, 32), jnp.float32),
                  jax.ShapeDtypeStruct((np_, 32), jnp.float32)),
        mesh=mesh,
        scratch_types=[
            pltpu.VMEM_SHARED((np_, 32), jnp.float32),
            pltpu.VMEM((3 * AGG_NB, CH), jnp.int32),
            pltpu.VMEM((3 * AGG_NB, CH), jnp.int32),
            pltpu.VMEM((2 * AGG_NB * CH, 32), jnp.float32),
            pltpu.SemaphoreType.DMA,
            pltpu.SemaphoreType.DMA,
            pltpu.SemaphoreType.DMA,
        ],
        compiler_params=pltpu.CompilerParams(use_tc_tiling_on_sc=False),
        name="sc_gcn_aggregate",
    )(hs_lo, hs_hi, src, dst)


def _pre_body(x_ref, p0_ref, p1_ref, We_ref, be_ref, W1_ref,
              h0_ref, lo_ref, hi_ref, dinv_ref):
    deg = 1.0 + p0_ref[:] + p1_ref[:]
    dinv = lax.rsqrt(deg)
    h0 = jax.nn.relu(jnp.dot(x_ref[:], We_ref[:],
                             preferred_element_type=jnp.float32) + be_ref[:])
    hs = dinv * jnp.dot(h0, W1_ref[:], preferred_element_type=jnp.float32)
    h0_ref[:] = h0
    lo_ref[:] = hs[:, :32]
    hi_ref[:] = hs[:, 32:]
    dinv_ref[:] = dinv


def _mid_body(lo_ref, hi_ref, dinv_ref, hprev_ref, b_ref, g_ref, bt_ref,
              Wn_ref, h_ref, nlo_ref, nhi_ref):
    accf = jnp.concatenate([lo_ref[:], hi_ref[:]], axis=1)
    dinv = dinv_ref[:]
    gcn = dinv * accf + b_ref[:]
    t = jax.nn.relu(g_ref[:] * (gcn * BN_SCALE) + bt_ref[:]) + hprev_ref[:]
    hs = dinv * jnp.dot(t, Wn_ref[:], preferred_element_type=jnp.float32)
    h_ref[:] = t
    nlo_ref[:] = hs[:, :32]
    nhi_ref[:] = hs[:, 32:]


def _final_body(lo_ref, hi_ref, dinv_ref, hprev_ref, b_ref, g_ref, bt_ref,
                Wf1_ref, bf1_ref, gf1_ref, btf1_ref, Wf2_ref, bf2_ref,
                out_ref):
    accf = jnp.concatenate([lo_ref[:], hi_ref[:]], axis=1)
    gcn = dinv_ref[:] * accf + b_ref[:]
    t = jax.nn.relu(g_ref[:] * (gcn * BN_SCALE) + bt_ref[:]) + hprev_ref[:]
    z = jnp.dot(t, Wf1_ref[:], preferred_element_type=jnp.float32) + bf1_ref[:]
    z = jax.nn.relu(gf1_ref[:] * (z * BN_SCALE) + btf1_ref[:])
    out_ref[:] = jnp.tanh(
        jnp.dot(z, Wf2_ref[:], preferred_element_type=jnp.float32) + bf2_ref[:])


def _row_spec(r, cols):
    return pl.BlockSpec((r, cols), lambda i: (i, 0))


def _full_spec(shape):
    return pl.BlockSpec(shape, lambda i: tuple(0 for _ in shape))


def _tc_pre(x, p0, p1, We, be, W1, *, n, np_, r):
    grid = (n // r,)
    return pl.pallas_call(
        _pre_body,
        grid=grid,
        in_specs=[_row_spec(r, 2), _row_spec(r, 1), _row_spec(r, 1),
                  _full_spec((2, 64)), _full_spec((1, 64)),
                  _full_spec((64, 64))],
        out_specs=[_row_spec(r, 64), _row_spec(r, 32), _row_spec(r, 32),
                   _row_spec(r, 1)],
        out_shape=[jax.ShapeDtypeStruct((n, 64), jnp.float32),
                   jax.ShapeDtypeStruct((np_, 32), jnp.float32),
                   jax.ShapeDtypeStruct((np_, 32), jnp.float32),
                   jax.ShapeDtypeStruct((n, 1), jnp.float32)],
        name="tc_gnn_pre",
    )(x, p0, p1, We, be, W1)


def _tc_mid(acc_lo, acc_hi, dinv, hprev, b, g, bt, Wn, *, n, np_, r):
    grid = (n // r,)
    return pl.pallas_call(
        _mid_body,
        grid=grid,
        in_specs=[_row_spec(r, 32), _row_spec(r, 32), _row_spec(r, 1),
                  _row_spec(r, 64), _full_spec((1, 64)), _full_spec((1, 64)),
                  _full_spec((1, 64)), _full_spec((64, 64))],
        out_specs=[_row_spec(r, 64), _row_spec(r, 32), _row_spec(r, 32)],
        out_shape=[jax.ShapeDtypeStruct((n, 64), jnp.float32),
                   jax.ShapeDtypeStruct((np_, 32), jnp.float32),
                   jax.ShapeDtypeStruct((np_, 32), jnp.float32)],
        name="tc_gnn_mid",
    )(acc_lo, acc_hi, dinv, hprev, b, g, bt, Wn)


def _tc_final(acc_lo, acc_hi, dinv, hprev, b, g, bt, Wf1, bf1, gf1, btf1,
              Wf2, bf2, *, n, r):
    grid = (n // r,)
    return pl.pallas_call(
        _final_body,
        grid=grid,
        in_specs=[_row_spec(r, 32), _row_spec(r, 32), _row_spec(r, 1),
                  _row_spec(r, 64), _full_spec((1, 64)), _full_spec((1, 64)),
                  _full_spec((1, 64)), _full_spec((64, 32)),
                  _full_spec((1, 32)), _full_spec((1, 32)),
                  _full_spec((1, 32)), _full_spec((32, 2)),
                  _full_spec((1, 2))],
        out_specs=[_row_spec(r, 2)],
        out_shape=[jax.ShapeDtypeStruct((n, 2), jnp.float32)],
        name="tc_gnn_final",
    )(acc_lo, acc_hi, dinv, hprev, b, g, bt, Wf1, bf1, gf1, btf1, Wf2, bf2)[0]


def kernel(x, edge_index, W_enc, b_enc, W1, b1, g1, bt1, W2, b2, g2, bt2,
           W3, b3, g3, bt3, Wf1, bf1, gf1, btf1, Wf2, bf2):
    n = x.shape[0]
    e = edge_index.shape[1]
    r = 5000 if n % 5000 == 0 else (1000 if n % 1000 == 0 else 8)
    np_ = -(-n // (NS * 8)) * (NS * 8)
    rps = np_ // NS

    per_w = -(-e // (CH * NC * NS * DEG_NB)) * DEG_NB
    e_pad = per_w * CH * NC * NS
    src = jnp.concatenate([edge_index[0], jnp.zeros((e_pad - e,), jnp.int32)])
    dst = jnp.concatenate([edge_index[1],
                           jnp.full((e_pad - e,), n, jnp.int32)])
    cps = e_pad // (CH * NS)

    pdeg = _sc_degree(dst, np_=np_, rps=rps, cpw=per_w)
    p0 = pdeg[:n].reshape(n, 1)
    p1 = pdeg[np_:np_ + n].reshape(n, 1)

    be = b_enc.reshape(1, 64)
    h0, lo, hi, dinv = _tc_pre(x, p0, p1, W_enc, be, W1, n=n, np_=np_, r=r)

    agg = functools.partial(_sc_aggregate, src=src, dst=dst,
                            np_=np_, rps=rps, cps=cps)

    a_lo, a_hi = agg(lo, hi)
    h1, lo, hi = _tc_mid(a_lo, a_hi, dinv, h0, b1.reshape(1, 64),
                         g1.reshape(1, 64), bt1.reshape(1, 64), W2,
                         n=n, np_=np_, r=r)
    a_lo, a_hi = agg(lo, hi)
    h2, lo, hi = _tc_mid(a_lo, a_hi, dinv, h1, b2.reshape(1, 64),
                         g2.reshape(1, 64), bt2.reshape(1, 64), W3,
                         n=n, np_=np_, r=r)
    a_lo, a_hi = agg(lo, hi)
    return _tc_final(a_lo, a_hi, dinv, h2, b3.reshape(1, 64),
                     g3.reshape(1, 64), bt3.reshape(1, 64), Wf1,
                     bf1.reshape(1, 32), gf1.reshape(1, 32),
                     btf1.reshape(1, 32), Wf2, bf2.reshape(1, 2), n=n, r=r)


# speedup vs baseline: 24.5085x; 1.2057x over previous
import functools
import math

import jax
import jax.numpy as jnp
from jax import lax
from jax.experimental import pallas as pl
from jax.experimental.pallas import tpu as pltpu
from jax.experimental.pallas import tpu_sc as plsc

NC = 2
NS = 16
CH = 128
BN_SCALE = 1.0 / math.sqrt(1.0 + 1e-5)


DEG_NB = 3
AGG_NB = 3


def _deg_body(np_, rps, cpw, dst_ref, out_ref, acc, idxd, ones, zbuf, semi,
              sems):
    nb = DEG_NB
    c = lax.axis_index("c")
    s = lax.axis_index("s")
    w = c * NS + s
    base = w * cpw
    grp = cpw // nb
    zlen = -(-rps // 16) * 16

    def fill_z(i, _):
        zbuf[pl.ds(i * 16, 16)] = jnp.zeros((16,), jnp.float32)
        return 0

    lax.fori_loop(0, zlen // 16, fill_z, 0)

    def fill_o(i, _):
        ones[pl.ds(i * 16, 16)] = jnp.ones((16,), jnp.float32)
        return 0

    lax.fori_loop(0, CH // 16, fill_o, 0)

    pltpu.sync_copy(zbuf.at[pl.ds(0, rps)], acc.at[pl.ds(s * rps, rps)])
    plsc.subcore_barrier()

    def idx_issue(g, po):
        for b in range(nb):
            ch = base + g * nb + b
            pltpu.async_copy(dst_ref.at[pl.ds(ch * CH, CH)], idxd.at[po + b],
                             semi)
        for b in range(nb):
            pltpu.make_async_copy(dst_ref.at[pl.ds(0, CH)], idxd.at[po + b],
                                  semi).wait()

    def scatter_issue(po):
        for b in range(nb):
            pltpu.async_copy(ones, acc.at[idxd.at[po + b]], sems, add=True)

    def scatter_wait(po):
        for b in range(nb):
            pltpu.make_async_copy(ones, acc.at[pl.ds(0, CH)], sems).wait()

    idx_issue(0, 0)

    def loop_body(t, _):
        po = (t % 2) * nb
        qo = nb - po

        @pl.when(t > 0)
        def _():
            scatter_wait(qo)

        scatter_issue(po)
        idx_issue(jnp.minimum(t + 1, grp - 1), qo)
        return 0

    lax.fori_loop(0, grp, loop_body, 0)
    scatter_wait(((grp - 1) % 2) * nb)
    plsc.subcore_barrier()
    pltpu.sync_copy(acc.at[pl.ds(s * rps, rps)],
                    out_ref.at[pl.ds(c * np_ + s * rps, rps)])


def _scatter_body(rps, cps, lo_ref, hi_ref, src_ref, dst_ref,
                  out_lo_ref, out_hi_ref, acc, idxs, idxd, rows,
                  semi, semg, sems):
    nb = AGG_NB
    c = lax.axis_index("c")
    s = lax.axis_index("s")
    grp = cps // nb

    def run(hs_ref, out_ref):
        base = s * cps
        pltpu.sync_copy(hs_ref.at[pl.ds(s * rps, rps)],
                        acc.at[pl.ds(s * rps, rps)])
        plsc.subcore_barrier()

        def idx_issue(g, so):
            for b in range(nb):
                ch = base + g * nb + b
                pltpu.async_copy(src_ref.at[pl.ds(ch * CH, CH)],
                                 idxs.at[so + b], semi)
                pltpu.async_copy(dst_ref.at[pl.ds(ch * CH, CH)],
                                 idxd.at[so + b], semi)

        def idx_wait(so):
            for b in range(nb):
                pltpu.make_async_copy(src_ref.at[pl.ds(0, CH)],
                                      idxs.at[so + b], semi).wait()
                pltpu.make_async_copy(src_ref.at[pl.ds(0, CH)],
                                      idxd.at[so + b], semi).wait()

        def gather_issue(po, so):
            for b in range(nb):
                pltpu.async_copy(hs_ref.at[idxs.at[so + b]],
                                 rows.at[pl.ds((po + b) * CH, CH)], semg)

        def gather_wait(po):
            for b in range(nb):
                pltpu.make_async_copy(hs_ref.at[pl.ds(0, CH)],
                                      rows.at[pl.ds((po + b) * CH, CH)],
                                      semg).wait()

        def scatter_issue(po, so):
            for b in range(nb):
                pltpu.async_copy(rows.at[pl.ds((po + b) * CH, CH)],
                                 acc.at[idxd.at[so + b]], sems, add=True)

        def scatter_wait(po):
            for b in range(nb):
                pltpu.make_async_copy(rows.at[pl.ds((po + b) * CH, CH)],
                                      acc.at[pl.ds(0, CH)], sems).wait()

        idx_issue(0, 0)
        idx_wait(0)
        gather_issue(0, 0)
        idx_issue(jnp.minimum(1, grp - 1), nb)

        def loop_body(t, _):
            po = (t % 2) * nb
            qo = nb - po
            so = (t % 3) * nb
            so1 = ((t + 1) % 3) * nb
            so2 = ((t + 2) % 3) * nb
            gather_wait(po)

            @pl.when(t > 0)
            def _():
                scatter_wait(qo)

            scatter_issue(po, so)
            idx_wait(so1)
            gather_issue(qo, so1)
            idx_issue(jnp.minimum(t + 2, grp - 1), so2)
            return 0

        lax.fori_loop(0, grp, loop_body, 0)
        gather_wait((grp % 2) * nb)
        scatter_wait(((grp - 1) % 2) * nb)
        idx_wait(((grp + 1) % 3) * nb)
        plsc.subcore_barrier()
        pltpu.sync_copy(acc.at[pl.ds(s * rps, rps)],
                        out_ref.at[pl.ds(s * rps, rps)])

    pl.when(c == 0)(lambda: run(lo_ref, out_lo_ref))
    pl.when(c == 1)(lambda: run(hi_ref, out_hi_ref))


@functools.partial(jax.jit, static_argnames=("np_", "rps", "cpw"))
def _sc_degree(dst, *, np_, rps, cpw):
    mesh = plsc.VectorSubcoreMesh(core_axis_name="c", subcore_axis_name="s")
    body = functools.partial(_deg_body, np_, rps, cpw)
    return pl.kernel(
        body,
        out_type=jax.ShapeDtypeStruct((NC * np_,), jnp.float32),
        mesh=mesh,
        scratch_types=[
            pltpu.VMEM_SHARED((np_,), jnp.float32),
            pltpu.VMEM((2 * DEG_NB, CH), jnp.int32),
            pltpu.VMEM((CH,), jnp.float32),
            pltpu.VMEM((-(-rps // 16) * 16,), jnp.float32),
            pltpu.SemaphoreType.DMA,
            pltpu.SemaphoreType.DMA,
        ],
        compiler_params=pltpu.CompilerParams(use_tc_tiling_on_sc=False),
        name="sc_gcn_degree",
    )(dst)


@functools.partial(jax.jit, static_argnames=("np_", "rps", "cps"))
def _sc_aggregate(hs_lo, hs_hi, src, dst, *, np_, rps, cps):
    mesh = plsc.VectorSubcoreMesh(core_axis_name="c", subcore_axis_name="s")
    body = functools.partial(_scatter_body, rps, cps)
    return pl.kernel(
        body,
        out_type=(jax.ShapeDtypeStruct((np_, 32), jnp.float32),
                  jax.ShapeDtypeStruct((np_, 32), jnp.float32)),
        mesh=mesh,
        scratch_types=[
            pltpu.VMEM_SHARED((np_, 32), jnp.float32),
            pltpu.VMEM((3 * AGG_NB, CH), jnp.int32),
            pltpu.VMEM((3 * AGG_NB, CH), jnp.int32),
            pltpu.VMEM((2 * AGG_NB * CH, 32), jnp.float32),
            pltpu.SemaphoreType.DMA,
            pltpu.SemaphoreType.DMA,
            pltpu.SemaphoreType.DMA,
        ],
        compiler_params=pltpu.CompilerParams(use_tc_tiling_on_sc=False),
        name="sc_gcn_aggregate",
    )(hs_lo, hs_hi, src, dst)


def _dinv_packed(p0, p1, rp):
    deg4 = 1.0 + p0 + p1
    dinv4 = lax.rsqrt(deg4)
    parts = [jnp.broadcast_to(dinv4[:, k:k + 1], (rp, 32)) for k in range(4)]
    return jnp.concatenate(parts, axis=1)


def _pre_body(rp, x_ref, p0_ref, p1_ref, We_ref, be_ref, W1_ref,
              h0lo_ref, h0hi_ref, lo_ref, hi_ref):
    dinv = _dinv_packed(p0_ref[:], p1_ref[:], rp)
    d2 = jnp.concatenate([dinv, dinv], axis=1)
    h0 = jax.nn.relu(jnp.dot(x_ref[:], We_ref[:],
                             preferred_element_type=jnp.float32) + be_ref[:])
    hs = d2 * jnp.dot(h0, W1_ref[:], preferred_element_type=jnp.float32)
    h0lo_ref[:] = h0[:, :128]
    h0hi_ref[:] = h0[:, 128:]
    lo_ref[:] = hs[:, :128]
    hi_ref[:] = hs[:, 128:]


def _mid_body(rp, lo_ref, hi_ref, p0_ref, p1_ref, hplo_ref, hphi_ref,
              b_ref, g_ref, bt_ref, Wn_ref, hlo_ref, hhi_ref,
              nlo_ref, nhi_ref):
    dinv = _dinv_packed(p0_ref[:], p1_ref[:], rp)
    d2 = jnp.concatenate([dinv, dinv], axis=1)
    accf = jnp.concatenate([lo_ref[:], hi_ref[:]], axis=1)
    hprev = jnp.concatenate([hplo_ref[:], hphi_ref[:]], axis=1)
    gcn = d2 * accf + b_ref[:]
    t = jax.nn.relu(g_ref[:] * (gcn * BN_SCALE) + bt_ref[:]) + hprev
    hs = d2 * jnp.dot(t, Wn_ref[:], preferred_element_type=jnp.float32)
    hlo_ref[:] = t[:, :128]
    hhi_ref[:] = t[:, 128:]
    nlo_ref[:] = hs[:, :128]
    nhi_ref[:] = hs[:, 128:]


def _final_body(rp, lo_ref, hi_ref, p0_ref, p1_ref, hplo_ref, hphi_ref,
                b_ref, g_ref, bt_ref, Wf1_ref, bf1_ref, gf1_ref, btf1_ref,
                Wf2_ref, bf2_ref, out_ref):
    dinv = _dinv_packed(p0_ref[:], p1_ref[:], rp)
    d2 = jnp.concatenate([dinv, dinv], axis=1)
    accf = jnp.concatenate([lo_ref[:], hi_ref[:]], axis=1)
    hprev = jnp.concatenate([hplo_ref[:], hphi_ref[:]], axis=1)
    gcn = d2 * accf + b_ref[:]
    t = jax.nn.relu(g_ref[:] * (gcn * BN_SCALE) + bt_ref[:]) + hprev
    z = jnp.dot(t, Wf1_ref[:], preferred_element_type=jnp.float32) + bf1_ref[:]
    z = jax.nn.relu(gf1_ref[:] * (z * BN_SCALE) + btf1_ref[:])
    out_ref[:] = jnp.tanh(
        jnp.dot(z, Wf2_ref[:], preferred_element_type=jnp.float32) + bf2_ref[:])


def _row_spec(r, cols):
    return pl.BlockSpec((r, cols), lambda i: (i, 0))


def _full_spec(shape):
    return pl.BlockSpec(shape, lambda i: tuple(0 for _ in shape))


def _tc_pre(x, p0, p1, We, be, W1, *, np_p, rp):
    grid = (np_p // rp,)
    return pl.pallas_call(
        functools.partial(_pre_body, rp),
        grid=grid,
        in_specs=[_row_spec(rp, 8), _row_spec(rp, 4), _row_spec(rp, 4),
                  _full_spec((8, 256)), _full_spec((1, 256)),
                  _full_spec((256, 256))],
        out_specs=[_row_spec(rp, 128)] * 4,
        out_shape=[jax.ShapeDtypeStruct((np_p, 128), jnp.float32)] * 4,
        name="tc_gnn_pre",
    )(x, p0, p1, We, be, W1)


def _tc_mid(acc_lo, acc_hi, p0, p1, hplo, hphi, b, g, bt, Wn, *, np_p,
            rp):
    grid = (np_p // rp,)
    return pl.pallas_call(
        functools.partial(_mid_body, rp),
        grid=grid,
        in_specs=[_row_spec(rp, 128), _row_spec(rp, 128),
                  _row_spec(rp, 4), _row_spec(rp, 4),
                  _row_spec(rp, 128), _row_spec(rp, 128),
                  _full_spec((1, 256)), _full_spec((1, 256)),
                  _full_spec((1, 256)), _full_spec((256, 256))],
        out_specs=[_row_spec(rp, 128)] * 4,
        out_shape=[jax.ShapeDtypeStruct((np_p, 128), jnp.float32)] * 4,
        name="tc_gnn_mid",
    )(acc_lo, acc_hi, p0, p1, hplo, hphi, b, g, bt, Wn)


def _tc_final(acc_lo, acc_hi, p0, p1, hplo, hphi, b, g, bt, Wf1, bf1, gf1,
              btf1, Wf2, bf2, *, np_p, rp):
    grid = (np_p // rp,)
    return pl.pallas_call(
        functools.partial(_final_body, rp),
        grid=grid,
        in_specs=[_row_spec(rp, 128), _row_spec(rp, 128),
                  _row_spec(rp, 4), _row_spec(rp, 4),
                  _row_spec(rp, 128), _row_spec(rp, 128),
                  _full_spec((1, 256)), _full_spec((1, 256)),
                  _full_spec((1, 256)), _full_spec((256, 128)),
                  _full_spec((1, 128)), _full_spec((1, 128)),
                  _full_spec((1, 128)), _full_spec((128, 8)),
                  _full_spec((1, 8))],
        out_specs=[_row_spec(rp, 8)],
        out_shape=[jax.ShapeDtypeStruct((np_p, 8), jnp.float32)],
        name="tc_gnn_final",
    )(acc_lo, acc_hi, p0, p1, hplo, hphi, b, g, bt, Wf1, bf1, gf1, btf1,
      Wf2, bf2)[0]


def kernel(x, edge_index, W_enc, b_enc, W1, b1, g1, bt1, W2, b2, g2, bt2,
           W3, b3, g3, bt3, Wf1, bf1, gf1, btf1, Wf2, bf2):
    n = x.shape[0]
    e = edge_index.shape[1]
    np_ = -(-n // (NS * 8)) * (NS * 8)
    rps = np_ // NS

    per_w = -(-e // (CH * NC * NS * DEG_NB)) * DEG_NB
    e_pad = per_w * CH * NC * NS
    src = jnp.concatenate([edge_index[0], jnp.zeros((e_pad - e,), jnp.int32)])
    dst = jnp.concatenate([edge_index[1],
                           jnp.full((e_pad - e,), n, jnp.int32)])
    cps = e_pad // (CH * NS)

    n_p, np_p = n // 4, np_ // 4
    rp = np_p // 4 if np_p % 32 == 0 else np_p

    pdeg = _sc_degree(dst, np_=np_, rps=rps, cpw=per_w)
    p0 = pdeg[:np_].reshape(np_p, 4)
    p1 = pdeg[np_:].reshape(np_p, 4)
    xp = jnp.concatenate([x.reshape(n_p, 8),
                          jnp.zeros((np_p - n_p, 8), jnp.float32)])

    i4 = jnp.eye(4, dtype=jnp.float32)

    def bd(m):
        return jnp.kron(i4, m)

    def wcat(w):
        return jnp.block([[bd(w[:32, :32]), bd(w[:32, 32:])],
                          [bd(w[32:, :32]), bd(w[32:, 32:])]])

    def vcat(v):
        return jnp.concatenate([jnp.tile(v[:32], 4),
                                jnp.tile(v[32:], 4)]).reshape(1, 256)

    wecat = jnp.concatenate([bd(W_enc[:, :32]), bd(W_enc[:, 32:])], axis=1)
    wf1cat = jnp.concatenate([bd(Wf1[:32, :]), bd(Wf1[32:, :])], axis=0)

    h0lo, h0hi, lo, hi = _tc_pre(xp, p0, p1, wecat, vcat(b_enc), wcat(W1),
                                 np_p=np_p, rp=rp)

    def agg(lo_p, hi_p):
        a_lo, a_hi = _sc_aggregate(lo_p.reshape(np_, 32),
                                   hi_p.reshape(np_, 32), src, dst,
                                   np_=np_, rps=rps, cps=cps)
        return a_lo.reshape(np_p, 128), a_hi.reshape(np_p, 128)

    a_lo, a_hi = agg(lo, hi)
    h1lo, h1hi, lo, hi = _tc_mid(a_lo, a_hi, p0, p1, h0lo, h0hi, vcat(b1),
                                 vcat(g1), vcat(bt1), wcat(W2),
                                 np_p=np_p, rp=rp)
    a_lo, a_hi = agg(lo, hi)
    h2lo, h2hi, lo, hi = _tc_mid(a_lo, a_hi, p0, p1, h1lo, h1hi, vcat(b2),
                                 vcat(g2), vcat(bt2), wcat(W3),
                                 np_p=np_p, rp=rp)
    a_lo, a_hi = agg(lo, hi)
    out_p = _tc_final(a_lo, a_hi, p0, p1, h2lo, h2hi, vcat(b3), vcat(g3),
                      vcat(bt3), wf1cat, jnp.tile(bf1, 4).reshape(1, 128),
                      jnp.tile(gf1, 4).reshape(1, 128),
                      jnp.tile(btf1, 4).reshape(1, 128), bd(Wf2),
                      jnp.tile(bf2, 4).reshape(1, 8), np_p=np_p, rp=rp)
    return out_p[:n_p].reshape(n, 2)


# speedup vs baseline: 24.5390x; 1.0012x over previous
import functools
import math

import jax
import jax.numpy as jnp
from jax import lax
from jax.experimental import pallas as pl
from jax.experimental.pallas import tpu as pltpu
from jax.experimental.pallas import tpu_sc as plsc

NC = 2
NS = 16
CH = 128
BN_SCALE = 1.0 / math.sqrt(1.0 + 1e-5)


DEG_NB = 3
AGG_NB = 3


def _deg_body(np_, rps, cpw, dst_ref, out_ref, acc, idxd, ones, zbuf, semi,
              sems):
    nb = DEG_NB
    c = lax.axis_index("c")
    s = lax.axis_index("s")
    w = c * NS + s
    base = w * cpw
    grp = cpw // nb
    zlen = -(-rps // 16) * 16

    def fill_z(i, _):
        zbuf[pl.ds(i * 16, 16)] = jnp.zeros((16,), jnp.float32)
        return 0

    lax.fori_loop(0, zlen // 16, fill_z, 0)

    def fill_o(i, _):
        ones[pl.ds(i * 16, 16)] = jnp.ones((16,), jnp.float32)
        return 0

    lax.fori_loop(0, CH // 16, fill_o, 0)

    pltpu.sync_copy(zbuf.at[pl.ds(0, rps)], acc.at[pl.ds(s * rps, rps)])
    plsc.subcore_barrier()

    def idx_issue(g, po):
        for b in range(nb):
            ch = base + g * nb + b
            pltpu.async_copy(dst_ref.at[pl.ds(ch * CH, CH)], idxd.at[po + b],
                             semi)
        for b in range(nb):
            pltpu.make_async_copy(dst_ref.at[pl.ds(0, CH)], idxd.at[po + b],
                                  semi).wait()

    def scatter_issue(po):
        for b in range(nb):
            pltpu.async_copy(ones, acc.at[idxd.at[po + b]], sems, add=True)

    def scatter_wait(po):
        for b in range(nb):
            pltpu.make_async_copy(ones, acc.at[pl.ds(0, CH)], sems).wait()

    idx_issue(0, 0)

    def loop_body(t, _):
        po = (t % 2) * nb
        qo = nb - po

        @pl.when(t > 0)
        def _():
            scatter_wait(qo)

        scatter_issue(po)
        idx_issue(jnp.minimum(t + 1, grp - 1), qo)
        return 0

    lax.fori_loop(0, grp, loop_body, 0)
    scatter_wait(((grp - 1) % 2) * nb)
    plsc.subcore_barrier()
    pltpu.sync_copy(acc.at[pl.ds(s * rps, rps)],
                    out_ref.at[pl.ds(c * np_ + s * rps, rps)])


def _scatter_body(rps, cps, lo_ref, hi_ref, src_ref, dst_ref,
                  out_lo_ref, out_hi_ref, acc, idxs, idxd, rows,
                  semi, semg, sems_a, sems_b):
    nb = AGG_NB
    c = lax.axis_index("c")
    s = lax.axis_index("s")
    grp = cps // nb

    def run(hs_ref, out_ref):
        base = s * cps
        pltpu.sync_copy(hs_ref.at[pl.ds(s * rps, rps)],
                        acc.at[pl.ds(s * rps, rps)])
        plsc.subcore_barrier()

        def idx_issue(g, so):
            for b in range(nb):
                ch = base + g * nb + b
                pltpu.async_copy(src_ref.at[pl.ds(ch * CH, CH)],
                                 idxs.at[so + b], semi)
                pltpu.async_copy(dst_ref.at[pl.ds(ch * CH, CH)],
                                 idxd.at[so + b], semi)

        def idx_wait(so):
            for b in range(nb):
                pltpu.make_async_copy(src_ref.at[pl.ds(0, CH)],
                                      idxs.at[so + b], semi).wait()
                pltpu.make_async_copy(src_ref.at[pl.ds(0, CH)],
                                      idxd.at[so + b], semi).wait()

        def gather_issue(po, so):
            for b in range(nb):
                pltpu.async_copy(hs_ref.at[idxs.at[so + b]],
                                 rows.at[pl.ds((po + b) * CH, CH)], semg)

        def gather_wait(po):
            for b in range(nb):
                pltpu.make_async_copy(hs_ref.at[pl.ds(0, CH)],
                                      rows.at[pl.ds((po + b) * CH, CH)],
                                      semg).wait()

        def scatter_issue(po, so, sem):
            for b in range(nb):
                pltpu.async_copy(rows.at[pl.ds((po + b) * CH, CH)],
                                 acc.at[idxd.at[so + b]], sem, add=True)

        def scatter_wait(po, sem):
            for b in range(nb):
                pltpu.make_async_copy(rows.at[pl.ds((po + b) * CH, CH)],
                                      acc.at[pl.ds(0, CH)], sem).wait()

        idx_issue(0, 0)
        idx_wait(0)
        gather_issue(0, 0)
        idx_issue(jnp.minimum(1, grp - 1), nb)

        def loop_body(u, _):
            for par in range(2):
                t = 2 * u + par
                po = par * nb
                qo = nb - po
                so = lax.rem(t, 3) * nb
                so1 = lax.rem(t + 1, 3) * nb
                so2 = lax.rem(t + 2, 3) * nb
                sem_cur = sems_a if par == 0 else sems_b
                sem_prev = sems_b if par == 0 else sems_a
                gather_wait(po)
                scatter_issue(po, so, sem_cur)
                if par == 0:
                    @pl.when(u > 0)
                    def _(qo=qo, sem_prev=sem_prev):
                        scatter_wait(qo, sem_prev)
                else:
                    scatter_wait(qo, sem_prev)
                idx_wait(so1)
                gather_issue(qo, so1)
                idx_issue(jnp.minimum(t + 2, grp - 1), so2)
            return 0

        lax.fori_loop(0, grp // 2, loop_body, 0)
        gather_wait((grp % 2) * nb)
        scatter_wait(((grp - 1) % 2) * nb,
                     sems_b if (grp - 1) % 2 else sems_a)
        idx_wait(((grp + 1) % 3) * nb)
        plsc.subcore_barrier()
        pltpu.sync_copy(acc.at[pl.ds(s * rps, rps)],
                        out_ref.at[pl.ds(s * rps, rps)])

    pl.when(c == 0)(lambda: run(lo_ref, out_lo_ref))
    pl.when(c == 1)(lambda: run(hi_ref, out_hi_ref))


@functools.partial(jax.jit, static_argnames=("np_", "rps", "cpw"))
def _sc_degree(dst, *, np_, rps, cpw):
    mesh = plsc.VectorSubcoreMesh(core_axis_name="c", subcore_axis_name="s")
    body = functools.partial(_deg_body, np_, rps, cpw)
    return pl.kernel(
        body,
        out_type=jax.ShapeDtypeStruct((NC * np_,), jnp.float32),
        mesh=mesh,
        scratch_types=[
            pltpu.VMEM_SHARED((np_,), jnp.float32),
            pltpu.VMEM((2 * DEG_NB, CH), jnp.int32),
            pltpu.VMEM((CH,), jnp.float32),
            pltpu.VMEM((-(-rps // 16) * 16,), jnp.float32),
            pltpu.SemaphoreType.DMA,
            pltpu.SemaphoreType.DMA,
        ],
        compiler_params=pltpu.CompilerParams(use_tc_tiling_on_sc=False),
        name="sc_gcn_degree",
    )(dst)


@functools.partial(jax.jit, static_argnames=("np_", "rps", "cps"))
def _sc_aggregate(hs_lo, hs_hi, src, dst, *, np_, rps, cps):
    mesh = plsc.VectorSubcoreMesh(core_axis_name="c", subcore_axis_name="s")
    body = functools.partial(_scatter_body, rps, cps)
    return pl.kernel(
        body,
        out_type=(jax.ShapeDtypeStruct((np_, 32), jnp.float32),
                  jax.ShapeDtypeStruct((np_, 32), jnp.float32)),
        mesh=mesh,
        scratch_types=[
            pltpu.VMEM_SHARED((np_, 32), jnp.float32),
            pltpu.VMEM((3 * AGG_NB, CH), jnp.int32),
            pltpu.VMEM((3 * AGG_NB, CH), jnp.int32),
            pltpu.VMEM((2 * AGG_NB * CH, 32), jnp.float32),
            pltpu.SemaphoreType.DMA,
            pltpu.SemaphoreType.DMA,
            pltpu.SemaphoreType.DMA,
            pltpu.SemaphoreType.DMA,
        ],
        compiler_params=pltpu.CompilerParams(use_tc_tiling_on_sc=False),
        name="sc_gcn_aggregate",
    )(hs_lo, hs_hi, src, dst)


def _dinv_packed(p0, p1, rp):
    deg4 = 1.0 + p0 + p1
    dinv4 = lax.rsqrt(deg4)
    parts = [jnp.broadcast_to(dinv4[:, k:k + 1], (rp, 32)) for k in range(4)]
    return jnp.concatenate(parts, axis=1)


def _pre_body(rp, x_ref, p0_ref, p1_ref, We_ref, be_ref, W1_ref,
              h0lo_ref, h0hi_ref, lo_ref, hi_ref):
    dinv = _dinv_packed(p0_ref[:], p1_ref[:], rp)
    d2 = jnp.concatenate([dinv, dinv], axis=1)
    h0 = jax.nn.relu(jnp.dot(x_ref[:], We_ref[:],
                             preferred_element_type=jnp.float32) + be_ref[:])
    hs = d2 * jnp.dot(h0, W1_ref[:], preferred_element_type=jnp.float32)
    h0lo_ref[:] = h0[:, :128]
    h0hi_ref[:] = h0[:, 128:]
    lo_ref[:] = hs[:, :128]
    hi_ref[:] = hs[:, 128:]


def _mid_body(rp, lo_ref, hi_ref, p0_ref, p1_ref, hplo_ref, hphi_ref,
              b_ref, g_ref, bt_ref, Wn_ref, hlo_ref, hhi_ref,
              nlo_ref, nhi_ref):
    dinv = _dinv_packed(p0_ref[:], p1_ref[:], rp)
    d2 = jnp.concatenate([dinv, dinv], axis=1)
    accf = jnp.concatenate([lo_ref[:], hi_ref[:]], axis=1)
    hprev = jnp.concatenate([hplo_ref[:], hphi_ref[:]], axis=1)
    gcn = d2 * accf + b_ref[:]
    t = jax.nn.relu(g_ref[:] * (gcn * BN_SCALE) + bt_ref[:]) + hprev
    hs = d2 * jnp.dot(t, Wn_ref[:], preferred_element_type=jnp.float32)
    hlo_ref[:] = t[:, :128]
    hhi_ref[:] = t[:, 128:]
    nlo_ref[:] = hs[:, :128]
    nhi_ref[:] = hs[:, 128:]


def _final_body(rp, lo_ref, hi_ref, p0_ref, p1_ref, hplo_ref, hphi_ref,
                b_ref, g_ref, bt_ref, Wf1_ref, bf1_ref, gf1_ref, btf1_ref,
                Wf2_ref, bf2_ref, out_ref):
    dinv = _dinv_packed(p0_ref[:], p1_ref[:], rp)
    d2 = jnp.concatenate([dinv, dinv], axis=1)
    accf = jnp.concatenate([lo_ref[:], hi_ref[:]], axis=1)
    hprev = jnp.concatenate([hplo_ref[:], hphi_ref[:]], axis=1)
    gcn = d2 * accf + b_ref[:]
    t = jax.nn.relu(g_ref[:] * (gcn * BN_SCALE) + bt_ref[:]) + hprev
    z = jnp.dot(t, Wf1_ref[:], preferred_element_type=jnp.float32) + bf1_ref[:]
    z = jax.nn.relu(gf1_ref[:] * (z * BN_SCALE) + btf1_ref[:])
    out_ref[:] = jnp.tanh(
        jnp.dot(z, Wf2_ref[:], preferred_element_type=jnp.float32) + bf2_ref[:])


def _row_spec(r, cols):
    return pl.BlockSpec((r, cols), lambda i: (i, 0))


def _full_spec(shape):
    return pl.BlockSpec(shape, lambda i: tuple(0 for _ in shape))


def _tc_pre(x, p0, p1, We, be, W1, *, np_p, rp):
    grid = (np_p // rp,)
    return pl.pallas_call(
        functools.partial(_pre_body, rp),
        grid=grid,
        in_specs=[_row_spec(rp, 8), _row_spec(rp, 4), _row_spec(rp, 4),
                  _full_spec((8, 256)), _full_spec((1, 256)),
                  _full_spec((256, 256))],
        out_specs=[_row_spec(rp, 128)] * 4,
        out_shape=[jax.ShapeDtypeStruct((np_p, 128), jnp.float32)] * 4,
        name="tc_gnn_pre",
    )(x, p0, p1, We, be, W1)


def _tc_mid(acc_lo, acc_hi, p0, p1, hplo, hphi, b, g, bt, Wn, *, np_p,
            rp):
    grid = (np_p // rp,)
    return pl.pallas_call(
        functools.partial(_mid_body, rp),
        grid=grid,
        in_specs=[_row_spec(rp, 128), _row_spec(rp, 128),
                  _row_spec(rp, 4), _row_spec(rp, 4),
                  _row_spec(rp, 128), _row_spec(rp, 128),
                  _full_spec((1, 256)), _full_spec((1, 256)),
                  _full_spec((1, 256)), _full_spec((256, 256))],
        out_specs=[_row_spec(rp, 128)] * 4,
        out_shape=[jax.ShapeDtypeStruct((np_p, 128), jnp.float32)] * 4,
        name="tc_gnn_mid",
    )(acc_lo, acc_hi, p0, p1, hplo, hphi, b, g, bt, Wn)


def _tc_final(acc_lo, acc_hi, p0, p1, hplo, hphi, b, g, bt, Wf1, bf1, gf1,
              btf1, Wf2, bf2, *, np_p, rp):
    grid = (np_p // rp,)
    return pl.pallas_call(
        functools.partial(_final_body, rp),
        grid=grid,
        in_specs=[_row_spec(rp, 128), _row_spec(rp, 128),
                  _row_spec(rp, 4), _row_spec(rp, 4),
                  _row_spec(rp, 128), _row_spec(rp, 128),
                  _full_spec((1, 256)), _full_spec((1, 256)),
                  _full_spec((1, 256)), _full_spec((256, 128)),
                  _full_spec((1, 128)), _full_spec((1, 128)),
                  _full_spec((1, 128)), _full_spec((128, 8)),
                  _full_spec((1, 8))],
        out_specs=[_row_spec(rp, 8)],
        out_shape=[jax.ShapeDtypeStruct((np_p, 8), jnp.float32)],
        name="tc_gnn_final",
    )(acc_lo, acc_hi, p0, p1, hplo, hphi, b, g, bt, Wf1, bf1, gf1, btf1,
      Wf2, bf2)[0]


def kernel(x, edge_index, W_enc, b_enc, W1, b1, g1, bt1, W2, b2, g2, bt2,
           W3, b3, g3, bt3, Wf1, bf1, gf1, btf1, Wf2, bf2):
    n = x.shape[0]
    e = edge_index.shape[1]
    np_ = -(-n // (NS * 8)) * (NS * 8)
    rps = np_ // NS

    per_w = -(-e // (CH * NC * NS * DEG_NB)) * DEG_NB
    e_pad = per_w * CH * NC * NS
    src = jnp.concatenate([edge_index[0], jnp.zeros((e_pad - e,), jnp.int32)])
    dst = jnp.concatenate([edge_index[1],
                           jnp.full((e_pad - e,), n, jnp.int32)])
    cps = e_pad // (CH * NS)

    n_p, np_p = n // 4, np_ // 4
    rp = np_p // 4 if np_p % 32 == 0 else np_p

    pdeg = _sc_degree(dst, np_=np_, rps=rps, cpw=per_w)
    p0 = pdeg[:np_].reshape(np_p, 4)
    p1 = pdeg[np_:].reshape(np_p, 4)
    xp = jnp.concatenate([x.reshape(n_p, 8),
                          jnp.zeros((np_p - n_p, 8), jnp.float32)])

    i4 = jnp.eye(4, dtype=jnp.float32)

    def bd(m):
        return jnp.kron(i4, m)

    def wcat(w):
        return jnp.block([[bd(w[:32, :32]), bd(w[:32, 32:])],
                          [bd(w[32:, :32]), bd(w[32:, 32:])]])

    def vcat(v):
        return jnp.concatenate([jnp.tile(v[:32], 4),
                                jnp.tile(v[32:], 4)]).reshape(1, 256)

    wecat = jnp.concatenate([bd(W_enc[:, :32]), bd(W_enc[:, 32:])], axis=1)
    wf1cat = jnp.concatenate([bd(Wf1[:32, :]), bd(Wf1[32:, :])], axis=0)

    h0lo, h0hi, lo, hi = _tc_pre(xp, p0, p1, wecat, vcat(b_enc), wcat(W1),
                                 np_p=np_p, rp=rp)

    def agg(lo_p, hi_p):
        a_lo, a_hi = _sc_aggregate(lo_p.reshape(np_, 32),
                                   hi_p.reshape(np_, 32), src, dst,
                                   np_=np_, rps=rps, cps=cps)
        return a_lo.reshape(np_p, 128), a_hi.reshape(np_p, 128)

    a_lo, a_hi = agg(lo, hi)
    h1lo, h1hi, lo, hi = _tc_mid(a_lo, a_hi, p0, p1, h0lo, h0hi, vcat(b1),
                                 vcat(g1), vcat(bt1), wcat(W2),
                                 np_p=np_p, rp=rp)
    a_lo, a_hi = agg(lo, hi)
    h2lo, h2hi, lo, hi = _tc_mid(a_lo, a_hi, p0, p1, h1lo, h1hi, vcat(b2),
                                 vcat(g2), vcat(bt2), wcat(W3),
                                 np_p=np_p, rp=rp)
    a_lo, a_hi = agg(lo, hi)
    out_p = _tc_final(a_lo, a_hi, p0, p1, h2lo, h2hi, vcat(b3), vcat(g3),
                      vcat(bt3), wf1cat, jnp.tile(bf1, 4).reshape(1, 128),
                      jnp.tile(gf1, 4).reshape(1, 128),
                      jnp.tile(btf1, 4).reshape(1, 128), bd(Wf2),
                      jnp.tile(bf2, 4).reshape(1, 8), np_p=np_p, rp=rp)
    return out_p[:n_p].reshape(n, 2)
